# baseline (speedup 1.0000x reference)
import functools
import math

import jax
import jax.numpy as jnp
import numpy as np
from jax import lax
from jax.experimental import pallas as pl
from jax.experimental.pallas import tpu as pltpu
from jax.experimental.pallas import tpu_sc as plsc

_N = 10000
_E = 320000
_IN1, _OUT1 = 256, 128
_IN2, _OUT2 = 128, 128
_HID = 128
_NLAYERS = 3
_MFEAT = 30
_KG = 10
_DN = 1.0 / math.sqrt(math.sqrt(128.0))

_BLK = 2000
_NB = _N // _BLK


def _rand_consts_raw():
    rk = jax.random.key(42)
    projs, egs = [], []
    for enc in (1, 2):
        key = jax.random.fold_in(rk, enc)
        for i in range(_NLAYERS):
            lk = jax.random.fold_in(key, i)
            kp, kg = jax.random.split(lk)
            k1, k2 = jax.random.split(kp)
            q, _ = jnp.linalg.qr(jax.random.normal(k1, (_HID, _HID)))
            mult = jnp.linalg.norm(jax.random.normal(k2, (_MFEAT, _HID)), axis=1)
            proj = mult[:, None] * q[:_MFEAT]
            g = -jnp.log(jax.random.exponential(kg, (_N, 1, _KG)))
            projs.append(proj.astype(jnp.float32))
            egs.append(jnp.exp(g).reshape(_N, _KG).astype(jnp.float32))
    return projs, egs


@functools.lru_cache(maxsize=None)
def _rand_consts_host():
    with jax.ensure_compile_time_eval(), \
         jax.default_device(jax.local_devices(backend="cpu")[0]):
        projs, egs = _rand_consts_raw()
        return ([np.asarray(p) for p in projs], [np.asarray(e) for e in egs])


def _rand_consts():
    try:
        return _rand_consts_host()
    except Exception:
        return _rand_consts_raw()


def _ln_elu(z, g, b):
    mu = jnp.mean(z, -1, keepdims=True)
    zc = z - mu
    var = jnp.mean(zc * zc, -1, keepdims=True)
    z = zc * lax.rsqrt(var + 1e-5) * g + b
    return jnp.where(z > 0, z, jnp.exp(jnp.minimum(z, 0.0)) - 1.0)


def _row_spec(width):
    return pl.BlockSpec((_BLK, width), lambda b, i: (b * _NB + i, 0))


def _wspec(shape):
    nz = len(shape)
    return pl.BlockSpec((1,) + shape, lambda b, i, nz=nz: (b,) + (0,) * nz)


def _prologue_body(x_ref, w_ref, b_ref, g_ref, bb_ref, o_ref):
    z = jnp.dot(x_ref[...], w_ref[0], preferred_element_type=jnp.float32)
    z = z + b_ref[0, 0]
    o_ref[...] = _ln_elu(z, g_ref[0, 0], bb_ref[0, 0])


def _prologue(x, w, b, g, bb):
    return pl.pallas_call(
        _prologue_body,
        grid=(2, _NB),
        in_specs=[
            _row_spec(x.shape[1]),
            _wspec(w.shape[1:]),
            _wspec((1, _HID)),
            _wspec((1, _HID)),
            _wspec((1, _HID)),
        ],
        out_specs=_row_spec(_HID),
        out_shape=jax.ShapeDtypeStruct((2 * _N, _HID), jnp.float32),
    )(x, w, b, g, bb)


def _stats_body(z_ref, eg_ref, wk_ref, bk_ref, wv_ref, bv_ref, pj_ref,
                s1_ref, t_ref, s1s_ref, ts_ref, gm_ref):
    i = pl.program_id(1)

    @pl.when(i == 0)
    def _():
        s1_ref[...] = jnp.zeros_like(s1_ref)
        t_ref[...] = jnp.zeros_like(t_ref)
        s1s_ref[...] = jnp.zeros_like(s1s_ref)
        ts_ref[...] = jnp.zeros_like(ts_ref)
        gm_ref[...] = jnp.full(gm_ref.shape, -jnp.inf, jnp.float32)

    z = z_ref[...]
    zk = (jnp.dot(z, wk_ref[0], preferred_element_type=jnp.float32)
          + bk_ref[0, 0]) * _DN
    dd = lax.dot_general(zk, pj_ref[0], (((1,), (1,)), ((), ())),
                         preferred_element_type=jnp.float32)
    diag = 0.5 * jnp.sum(zk * zk, -1, keepdims=True)
    p = jnp.exp(dd - diag)
    v = jnp.dot(z, wv_ref[0], preferred_element_type=jnp.float32) + bv_ref[0, 0]
    eg = eg_ref[...]
    for k in range(_KG):
        a = p * eg[:, k:k + 1]
        s1k = lax.dot_general(a, v, (((0,), (0,)), ((), ())),
                              preferred_element_type=jnp.float32)
        s1_ref[0, k] += s1k
    t_ref[0] += lax.dot_general(eg, v, (((0,), (0,)), ((), ())),
                                preferred_element_type=jnp.float32)
    s1s_ref[0] += lax.dot_general(eg, p, (((0,), (0,)), ((), ())),
                                  preferred_element_type=jnp.float32)
    ts_ref[...] += jnp.sum(eg, axis=0).reshape(1, 1, _KG)
    gm_ref[...] = jnp.maximum(gm_ref[...], jnp.reshape(jnp.max(dd), (1, 1, 1)))


def _stats(z, eg, wk, bk, wv, bv, pj):
    acc = lambda shape: pl.BlockSpec((1,) + shape,
                                     lambda b, i: tuple([b] + [0] * len(shape)))
    return pl.pallas_call(
        _stats_body,
        grid=(2, _NB),
        in_specs=[
            _row_spec(_HID),
            _row_spec(_KG),
            _wspec((_HID, _HID)),
            _wspec((1, _HID)),
            _wspec((_HID, _HID)),
            _wspec((1, _HID)),
            _wspec((_MFEAT, _HID)),
        ],
        out_specs=[acc((_KG, _MFEAT, _HID)), acc((_KG, _HID)),
                   acc((_KG, _MFEAT)), acc((1, _KG)), acc((1, 1))],
        out_shape=[
            jax.ShapeDtypeStruct((2, _KG, _MFEAT, _HID), jnp.float32),
            jax.ShapeDtypeStruct((2, _KG, _HID), jnp.float32),
            jax.ShapeDtypeStruct((2, _KG, _MFEAT), jnp.float32),
            jax.ShapeDtypeStruct((2, 1, _KG), jnp.float32),
            jax.ShapeDtypeStruct((2, 1, 1), jnp.float32),
        ],
    )(z, eg, wk, bk, wv, bv, pj)


def _apply_body(z_ref, wq_ref, bq_ref, pj_ref, s1_ref, t_ref, s1s_ref, ts_ref,
                gm_ref, wo_ref, bo_ref, lg_ref, lb_ref, o_ref):
    z = z_ref[...]
    zq = (jnp.dot(z, wq_ref[0], preferred_element_type=jnp.float32)
          + bq_ref[0, 0]) * _DN
    dd = lax.dot_general(zq, pj_ref[0], (((1,), (1,)), ((), ())),
                         preferred_element_type=jnp.float32)
    diag = 0.5 * jnp.sum(zq * zq, -1, keepdims=True)
    rmax = jnp.max(dd, -1, keepdims=True)
    qp = jnp.exp(dd - diag - rmax) + 1e-6
    em = jnp.exp(-gm_ref[0, 0, 0])
    acc = jnp.zeros((z.shape[0], _HID), jnp.float32)
    for k in range(_KG):
        kvsk = em * s1_ref[0, k] + 1e-6 * t_ref[0, k][None, :]
        kssk = em * s1s_ref[0, k] + 1e-6 * ts_ref[0, 0, k]
        num = jnp.dot(qp, kvsk, preferred_element_type=jnp.float32)
        den = jnp.sum(qp * kssk[None, :], -1, keepdims=True)
        acc = acc + num / den
    z_att = acc * (1.0 / _KG)
    out = jnp.dot(z_att, wo_ref[0], preferred_element_type=jnp.float32)
    out = out + bo_ref[0, 0] + z
    o_ref[...] = _ln_elu(out, lg_ref[0, 0], lb_ref[0, 0])


def _apply(z, wq, bq, pj, s1, t, s1s, ts, gm, wo, bo, lg, lb):
    acc = lambda shape: pl.BlockSpec((1,) + shape,
                                     lambda b, i: tuple([b] + [0] * len(shape)))
    return pl.pallas_call(
        _apply_body,
        grid=(2, _NB),
        in_specs=[
            _row_spec(_HID),
            _wspec((_HID, _HID)), _wspec((1, _HID)), _wspec((_MFEAT, _HID)),
            acc((_KG, _MFEAT, _HID)), acc((_KG, _HID)), acc((_KG, _MFEAT)),
            acc((1, _KG)), acc((1, 1)),
            _wspec((_HID, _HID)), _wspec((1, _HID)),
            _wspec((1, _HID)), _wspec((1, _HID)),
        ],
        out_specs=_row_spec(_HID),
        out_shape=jax.ShapeDtypeStruct((2 * _N, _HID), jnp.float32),
    )(z, wq, bq, pj, s1, t, s1s, ts, gm, wo, bo, lg, lb)


def _epilogue_body(z0_ref, z1_ref, z2_ref, z3_ref, w_ref, b_ref, o_ref):
    w = w_ref[0]
    out = jnp.dot(z0_ref[...], w[0:128], preferred_element_type=jnp.float32)
    out += jnp.dot(z1_ref[...], w[128:256], preferred_element_type=jnp.float32)
    out += jnp.dot(z2_ref[...], w[256:384], preferred_element_type=jnp.float32)
    out += jnp.dot(z3_ref[...], w[384:512], preferred_element_type=jnp.float32)
    o_ref[...] = out + b_ref[0, 0]


def _epilogue(z0, z1, z2, z3, w, b):
    return pl.pallas_call(
        _epilogue_body,
        grid=(2, _NB),
        in_specs=[_row_spec(_HID)] * 4 + [_wspec((4 * _HID, _HID)),
                                          _wspec((1, _HID))],
        out_specs=_row_spec(_HID),
        out_shape=jax.ShapeDtypeStruct((2 * _N, _HID), jnp.float32),
    )(z0, z1, z2, z3, w, b)


def _deg_of(degp_blk):
    return degp_blk[0, :, 0] + degp_blk[1, :, 0] + 1.0


_PSPEC = pl.BlockSpec((1, _BLK, _HID), lambda p, i: (p, i, 0))
_DEGSPEC2 = pl.BlockSpec((2, _BLK, 16), lambda p, i: (0, i, 0))
_DEGSPEC1 = pl.BlockSpec((2, _BLK, 16), lambda i: (0, i, 0))


def _gcn_pre_body(e_ref, degp_ref, w_ref, o_ref):
    dinv = lax.rsqrt(_deg_of(degp_ref[...]))
    xw = jnp.dot(e_ref[...], w_ref[0], preferred_element_type=jnp.float32)
    o_ref[0] = dinv[:, None] * xw


def _gcn_pre(e, degp, w):
    return pl.pallas_call(
        _gcn_pre_body,
        grid=(2, _NB),
        in_specs=[_row_spec(_HID), _DEGSPEC2, _wspec((_HID, _HID))],
        out_specs=_PSPEC,
        out_shape=jax.ShapeDtypeStruct((2, _N, _HID), jnp.float32),
    )(e, degp, w)


def _gcn_mid_body(acc_ref, xs_ref, degp_ref, b_ref, a_ref, w2_ref, o_ref):
    dinv = lax.rsqrt(_deg_of(degp_ref[...]))
    h = dinv[:, None] * (acc_ref[0] + xs_ref[0]) + b_ref[0, 0]
    h = jnp.where(h >= 0, h, a_ref[0, 0] * h)
    o_ref[0] = dinv[:, None] * jnp.dot(h, w2_ref[0],
                                       preferred_element_type=jnp.float32)


def _gcn_mid(acc, xs, degp, b, a, w2):
    return pl.pallas_call(
        _gcn_mid_body,
        grid=(2, _NB),
        in_specs=[_PSPEC, _PSPEC, _DEGSPEC2, _wspec((1, _HID)), _wspec((1, _HID)),
                  _wspec((_HID, _HID))],
        out_specs=_PSPEC,
        out_shape=jax.ShapeDtypeStruct((2, _N, _HID), jnp.float32),
    )(acc, xs, degp, b, a, w2)


def _combine_body(acc_ref, xs_ref, degp_ref, b_ref, a_ref, al_ref, o_ref):
    dinv = lax.rsqrt(_deg_of(degp_ref[...]))
    gs = []
    for p in (0, 1):
        g = dinv[:, None] * (acc_ref[p] + xs_ref[p]) + b_ref[...][p]
        g = jnp.where(g >= 0, g, a_ref[...][p] * g)
        nrm = jnp.sqrt(jnp.sum(g * g, -1, keepdims=True))
        gs.append(g / jnp.maximum(nrm, 1e-12))
    a0 = al_ref[0, 0]
    a1 = al_ref[0, 1]
    m = jnp.maximum(a0, a1)
    e0 = jnp.exp(a0 - m)
    e1 = jnp.exp(a1 - m)
    o_ref[...] = (e0 / (e0 + e1)) * gs[0] + (e1 / (e0 + e1)) * gs[1]


def _combine(acc2, xs2, degp, b2, a2, alpha):
    full2 = pl.BlockSpec((2, _BLK, _HID), lambda i: (0, i, 0))
    return pl.pallas_call(
        _combine_body,
        grid=(_NB,),
        in_specs=[full2, full2, _DEGSPEC1,
                  pl.BlockSpec((2, _HID), lambda i: (0, 0)),
                  pl.BlockSpec((2, _HID), lambda i: (0, 0)),
                  pl.BlockSpec((1, 2), lambda i: (0, 0))],
        out_specs=pl.BlockSpec((_BLK, _HID), lambda i: (i, 0)),
        out_shape=jax.ShapeDtypeStruct((_N, _HID), jnp.float32),
    )(acc2, xs2, degp, b2, a2, alpha)


def _dec_body(z_ref, w_ref, o_ref):
    o_ref[0] = jnp.dot(z_ref[...], w_ref[0], preferred_element_type=jnp.float32)


def _dec_mm(z, w3):
    return pl.pallas_call(
        _dec_body,
        grid=(3, _NB),
        in_specs=[pl.BlockSpec((_BLK, _HID), lambda t, i: (i, 0)),
                  pl.BlockSpec((1, _HID, _HID), lambda t, i: (t, 0, 0))],
        out_specs=pl.BlockSpec((1, _BLK, _HID), lambda t, i: (t, i, 0)),
        out_shape=jax.ShapeDtypeStruct((3, _N, _HID), jnp.float32),
    )(z, w3)


def _addp_body(p_ref, o_ref):
    o_ref[...] = p_ref[0] + p_ref[1]


def _add_partials(parts):
    return pl.pallas_call(
        _addp_body,
        grid=(_NB,),
        in_specs=[pl.BlockSpec((2, _BLK, _HID), lambda i: (0, i, 0))],
        out_specs=pl.BlockSpec((_BLK, _HID), lambda i: (i, 0)),
        out_shape=jax.ShapeDtypeStruct((_N, _HID), jnp.float32),
    )(parts)


_CH = 80
_NROW_T = 632
_NPAD = 16 * _NROW_T
_ZR = _NROW_T


def _zero_rows(zb, d):
    zeros16 = jnp.zeros((16,), jnp.float32)

    def zrow(r, c):
        for cc in range(d // 16):
            zb[r, pl.ds(cc * 16, 16)] = zeros16
        return c

    lax.fori_loop(0, _ZR, zrow, 0)


def _zero_acc_and_barrier(zb, acc, row0):
    pltpu.sync_copy(zb, acc.at[pl.ds(row0, _ZR)])
    plsc.subcore_barrier()


def _make_scatter(d, tpc):
    pt = _E // 16
    nch = pt // _CH

    @functools.partial(
        pl.kernel,
        out_type=jax.ShapeDtypeStruct((2 * tpc, _NPAD, d), jnp.float32),
        mesh=plsc.VectorSubcoreMesh(core_axis_name="c", subcore_axis_name="s"),
        compiler_params=pltpu.CompilerParams(use_tc_tiling_on_sc=False),
        scratch_types=[
            pltpu.VMEM((nch, _CH), jnp.int32),
            pltpu.VMEM((nch, _CH), jnp.int32),
            pltpu.VMEM((nch, _CH), jnp.float32),
            pltpu.VMEM((2, _CH, d), jnp.float32),
            pltpu.VMEM((_ZR, d), jnp.float32),
            pltpu.VMEM_SHARED((_NPAD, d), jnp.float32),
            pltpu.SemaphoreType.DMA,
            pltpu.SemaphoreType.DMA,
        ],
    )
    def k(*args):
        tbls = args[:2 * tpc]
        (gi_h, si_h, val_h, out_h,
         gi_v, si_v, val_v, rows_v, zb, acc, sem0, sem1) = args[2 * tpc:]
        sems = (sem0, sem1)
        cid = lax.axis_index("c")
        sid = lax.axis_index("s")
        row0 = sid * _NROW_T
        _zero_rows(zb, d)
        pltpu.sync_copy(gi_h.at[pl.ds(sid * nch, nch)], gi_v)
        pltpu.sync_copy(si_h.at[pl.ds(sid * nch, nch)], si_v)
        pltpu.sync_copy(val_h.at[pl.ds(sid * nch, nch)], val_v)

        def run(tbl, tglob):
            pltpu.sync_copy(zb, acc.at[pl.ds(row0, _ZR)])
            plsc.subcore_barrier()

            pltpu.async_copy(tbl.at[gi_v.at[0]], rows_v.at[0], sems[0])

            def body(jh, c):
                for b in range(2):
                    jj = jh * 2 + b
                    pltpu.make_async_copy(tbl.at[gi_v.at[jj]],
                                          rows_v.at[b], sems[b]).wait()

                    def sgrp(g, c2, b=b):
                        v16 = val_v[jj, pl.ds(g * 16, 16)]
                        for r in range(16):
                            s = v16[r]
                            for cc in range(d // 16):
                                sl = pl.ds(cc * 16, 16)
                                rows_v[b, g * 16 + r, sl] = \
                                    rows_v[b, g * 16 + r, sl] * s
                        return c2

                    lax.fori_loop(0, _CH // 16, sgrp, 0)
                    pltpu.async_copy(rows_v.at[b], acc.at[si_v.at[jj]],
                                     sems[b], add=True)

                    if b == 0:
                        @pl.when(jj == 0)
                        def _():
                            pltpu.async_copy(tbl.at[gi_v.at[1]],
                                             rows_v.at[1], sems[1])

                    @pl.when((jj >= 1) & (jj < nch - 1))
                    def _():
                        pltpu.make_async_copy(rows_v.at[1 - b],
                                              acc.at[si_v.at[jj]],
                                              sems[1 - b]).wait()
                        pltpu.async_copy(tbl.at[gi_v.at[jj + 1]],
                                         rows_v.at[1 - b], sems[1 - b])
                return c

            lax.fori_loop(0, nch // 2, body, 0)
            for b in range(2):
                pltpu.make_async_copy(rows_v.at[b], acc.at[si_v.at[0]],
                                      sems[b]).wait()
            plsc.subcore_barrier()
            pltpu.sync_copy(acc.at[pl.ds(row0, _ZR)],
                            out_h.at[tglob, pl.ds(row0, _ZR)])

        for tloc in range(tpc):
            @pl.when(cid == 0)
            def _(tloc=tloc):
                run(tbls[tloc], tloc)

            @pl.when(cid == 1)
            def _(tloc=tloc):
                run(tbls[tpc + tloc], tpc + tloc)

    return k


def _make_deg_kernel():
    @functools.partial(
        pl.kernel,
        out_type=jax.ShapeDtypeStruct((2, _NPAD, 16), jnp.float32),
        mesh=plsc.VectorSubcoreMesh(core_axis_name="c", subcore_axis_name="s"),
        compiler_params=pltpu.CompilerParams(use_tc_tiling_on_sc=False),
        scratch_types=[
            pltpu.VMEM((_CH,), jnp.int32),
            pltpu.VMEM((_CH, 16), jnp.float32),
            pltpu.VMEM((_ZR, 16), jnp.float32),
            pltpu.VMEM_SHARED((_NPAD, 16), jnp.float32),
        ],
    )
    def _deg_kernel(dst_h, out_h, dst_v, ones_v, zb, acc):
        cid = lax.axis_index("c")
        sid = lax.axis_index("s")
        row0 = sid * _NROW_T
        ones16 = jnp.ones((16,), jnp.float32)

        def orow(r, c):
            ones_v[r, pl.ds(0, 16)] = ones16
            return c

        lax.fori_loop(0, _CH, orow, 0)
        _zero_rows(zb, 16)
        _zero_acc_and_barrier(zb, acc, row0)

        pt = _E // 32
        nch = pt // _CH

        def body(j, c):
            base = (cid * 16 + sid) * pt + j * _CH
            pltpu.sync_copy(dst_h.at[pl.ds(base, _CH)], dst_v)
            pltpu.sync_copy(ones_v, acc.at[dst_v], add=True)
            return c

        lax.fori_loop(0, nch, body, 0)
        plsc.subcore_barrier()
        pltpu.sync_copy(acc.at[pl.ds(row0, _ZR)],
                        out_h.at[cid, pl.ds(row0, _ZR)])

    return _deg_kernel


_SC_CACHE = {}


def _get_sc(name):
    if name not in _SC_CACHE:
        _SC_CACHE['deg'] = _make_deg_kernel()
        _SC_CACHE['conv'] = _make_scatter(32, tpc=4)
        _SC_CACHE['dec'] = _make_scatter(32, tpc=6)
    return _SC_CACHE[name]


def _stk1(arrs):
    return jnp.stack(arrs)[:, None, :]

def _encode(x, p, projs, egs):
    w0 = jnp.stack([jnp.pad(pe['fc0_w'], ((0, _IN1 - pe['fc0_w'].shape[0]),
                                          (0, 0))) for pe in p])
    b0 = _stk1([pe['fc0_b'] for pe in p])
    g0 = _stk1([pe['ln0_g'] for pe in p])
    bb0 = _stk1([pe['ln0_b'] for pe in p])
    z = _prologue(x, w0, b0, g0, bb0)
    layers = [z]
    for i in range(_NLAYERS):
        cs = [pe['conv%d' % i] for pe in p]
        pj = jnp.stack([projs[0][i], projs[1][i]])
        eg = jnp.concatenate([egs[0][i], egs[1][i]], axis=0)
        wk = jnp.stack([c['Wk_w'] for c in cs])
        bk = _stk1([c['Wk_b'] for c in cs])
        wv = jnp.stack([c['Wv_w'] for c in cs])
        bv = _stk1([c['Wv_b'] for c in cs])
        s1, t, s1s, ts, gm = _stats(z, eg, wk, bk, wv, bv, pj)
        wq = jnp.stack([c['Wq_w'] for c in cs])
        bq = _stk1([c['Wq_b'] for c in cs])
        wo = jnp.stack([c['Wo_w'] for c in cs])
        bo = _stk1([c['Wo_b'] for c in cs])
        lg = _stk1([pe['ln%d_g' % (i + 1)] for pe in p])
        lb = _stk1([pe['ln%d_b' % (i + 1)] for pe in p])
        z = _apply(z, wq, bq, pj, s1, t, s1s, ts, gm, wo, bo, lg, lb)
        layers.append(z)
    w1 = jnp.stack([pe['fc1_w'] for pe in p])
    b1 = _stk1([pe['fc1_b'] for pe in p])
    return _epilogue(layers[0], layers[1], layers[2], layers[3], w1, b1)


def kernel(x1, x2, edge_index, adj_values, params):
    projs_all, egs_all = _rand_consts()
    projs = (projs_all[:_NLAYERS], projs_all[_NLAYERS:])
    egs = (egs_all[:_NLAYERS], egs_all[_NLAYERS:])

    src = edge_index[0]
    dst = edge_index[1]
    src2 = src.reshape(_E // _CH, _CH)
    dst2 = dst.reshape(_E // _CH, _CH)
    adj2 = adj_values.reshape(_E // _CH, _CH)

    degp = _get_sc('deg')(dst)

    x = jnp.concatenate(
        [x1, jnp.pad(x2, ((0, 0), (0, _IN1 - _IN2)))], axis=0)
    e = _encode(x, (params['enc1'], params['enc2']), projs, egs)

    f = params['fus']
    sc_conv = _get_sc('conv')
    sc_dec = _get_sc('dec')
    ones2 = jnp.ones((_E // _CH, _CH), jnp.float32)

    def split32(a):
        return a.reshape(2, _N, 4, 32).transpose(0, 2, 1, 3).reshape(8, _N, 32)

    def join32(a8):
        return a8.reshape(2, 4, _NPAD, 32).transpose(0, 2, 1, 3).reshape(
            2, _NPAD, 128)

    w12 = jnp.stack([f['c1_w'], f['c2_w']])
    xs = _gcn_pre(e, degp, w12)
    xs8 = split32(xs)
    acc1 = join32(sc_conv(*(xs8[i] for i in range(8)), src2, dst2, ones2))

    b12 = _stk1([f['c1_b'], f['c2_b']])
    a13 = _stk1([f['prelu1'], f['prelu3']])
    w34 = jnp.stack([f['c3_w'], f['c4_w']])
    xs2 = _gcn_mid(acc1, xs, degp, b12, a13, w34)
    xs28 = split32(xs2)
    acc2 = join32(sc_conv(*(xs28[i] for i in range(8)), src2, dst2, ones2))

    b34 = jnp.stack([f['c3_b'], f['c4_b']])
    a24 = jnp.stack([f['prelu2'], f['prelu4']])
    z = _combine(acc2, xs2, degp, b34, a24, f['alpha'].reshape(1, 2))

    w3 = jnp.stack([params['dec1_w'][:, :128], params['dec1_w'][:, 128:],
                    params['dec2_w']])
    zd = _dec_mm(z, w3)
    zd12 = zd.reshape(3, _N, 4, 32).transpose(0, 2, 1, 3).reshape(12, _N, 32)

    accd = sc_dec(*(zd12[i] for i in range(12)), dst2, src2, adj2)
    r1 = jnp.concatenate([accd[i, :_N] for i in range(8)], axis=1)
    r2 = jnp.concatenate([accd[i, :_N] for i in range(8, 12)], axis=1)

    return (z, r1, r2)


# speedup vs baseline: 1.4462x; 1.4462x over previous
import functools
import math

import jax
import jax.numpy as jnp
import numpy as np
from jax import lax
from jax.experimental import pallas as pl
from jax.experimental.pallas import tpu as pltpu
from jax.experimental.pallas import tpu_sc as plsc

_N = 10000
_E = 320000
_IN1, _OUT1 = 256, 128
_IN2, _OUT2 = 128, 128
_HID = 128
_NLAYERS = 3
_MFEAT = 30
_KG = 10
_DN = 1.0 / math.sqrt(math.sqrt(128.0))

_BLK = 2000
_NB = _N // _BLK


def _rand_consts_raw():
    rk = jax.random.key(42)
    projs, egs = [], []
    for enc in (1, 2):
        key = jax.random.fold_in(rk, enc)
        for i in range(_NLAYERS):
            lk = jax.random.fold_in(key, i)
            kp, kg = jax.random.split(lk)
            k1, k2 = jax.random.split(kp)
            q, _ = jnp.linalg.qr(jax.random.normal(k1, (_HID, _HID)))
            mult = jnp.linalg.norm(jax.random.normal(k2, (_MFEAT, _HID)), axis=1)
            proj = mult[:, None] * q[:_MFEAT]
            g = -jnp.log(jax.random.exponential(kg, (_N, 1, _KG)))
            projs.append(proj.astype(jnp.float32))
            egs.append(jnp.exp(g).reshape(_N, _KG).astype(jnp.float32))
    return projs, egs


@functools.lru_cache(maxsize=None)
def _rand_consts_host():
    with jax.ensure_compile_time_eval(), \
         jax.default_device(jax.local_devices(backend="cpu")[0]):
        projs, egs = _rand_consts_raw()
        return ([np.asarray(p) for p in projs], [np.asarray(e) for e in egs])


def _rand_consts():
    try:
        return _rand_consts_host()
    except Exception:
        return _rand_consts_raw()


def _ln_elu(z, g, b):
    mu = jnp.mean(z, -1, keepdims=True)
    zc = z - mu
    var = jnp.mean(zc * zc, -1, keepdims=True)
    z = zc * lax.rsqrt(var + 1e-5) * g + b
    return jnp.where(z > 0, z, jnp.exp(jnp.minimum(z, 0.0)) - 1.0)


def _row_spec(width):
    return pl.BlockSpec((_BLK, width), lambda b, i: (b * _NB + i, 0))


def _wspec(shape):
    nz = len(shape)
    return pl.BlockSpec((1,) + shape, lambda b, i, nz=nz: (b,) + (0,) * nz)


def _prologue_body(x_ref, w_ref, b_ref, g_ref, bb_ref, o_ref):
    z = jnp.dot(x_ref[...], w_ref[0], preferred_element_type=jnp.float32)
    z = z + b_ref[0, 0]
    o_ref[...] = _ln_elu(z, g_ref[0, 0], bb_ref[0, 0])


def _prologue(x, w, b, g, bb):
    return pl.pallas_call(
        _prologue_body,
        grid=(2, _NB),
        in_specs=[
            _row_spec(x.shape[1]),
            _wspec(w.shape[1:]),
            _wspec((1, _HID)),
            _wspec((1, _HID)),
            _wspec((1, _HID)),
        ],
        out_specs=_row_spec(_HID),
        out_shape=jax.ShapeDtypeStruct((2 * _N, _HID), jnp.float32),
    )(x, w, b, g, bb)


def _stats_body(z_ref, eg_ref, wk_ref, bk_ref, wv_ref, bv_ref, pj_ref,
                s1_ref, t_ref, s1s_ref, ts_ref, gm_ref):
    i = pl.program_id(1)

    @pl.when(i == 0)
    def _():
        s1_ref[...] = jnp.zeros_like(s1_ref)
        t_ref[...] = jnp.zeros_like(t_ref)
        s1s_ref[...] = jnp.zeros_like(s1s_ref)
        ts_ref[...] = jnp.zeros_like(ts_ref)
        gm_ref[...] = jnp.full(gm_ref.shape, -jnp.inf, jnp.float32)

    z = z_ref[...]
    zk = (jnp.dot(z, wk_ref[0], preferred_element_type=jnp.float32)
          + bk_ref[0, 0]) * _DN
    dd = lax.dot_general(zk, pj_ref[0], (((1,), (1,)), ((), ())),
                         preferred_element_type=jnp.float32)
    diag = 0.5 * jnp.sum(zk * zk, -1, keepdims=True)
    p = jnp.exp(dd - diag)
    v = jnp.dot(z, wv_ref[0], preferred_element_type=jnp.float32) + bv_ref[0, 0]
    eg = eg_ref[...]
    for k in range(_KG):
        a = p * eg[:, k:k + 1]
        s1k = lax.dot_general(a, v, (((0,), (0,)), ((), ())),
                              preferred_element_type=jnp.float32)
        s1_ref[0, k] += s1k
    t_ref[0] += lax.dot_general(eg, v, (((0,), (0,)), ((), ())),
                                preferred_element_type=jnp.float32)
    s1s_ref[0] += lax.dot_general(eg, p, (((0,), (0,)), ((), ())),
                                  preferred_element_type=jnp.float32)
    ts_ref[...] += jnp.sum(eg, axis=0).reshape(1, 1, _KG)
    gm_ref[...] = jnp.maximum(gm_ref[...], jnp.reshape(jnp.max(dd), (1, 1, 1)))


def _stats(z, eg, wk, bk, wv, bv, pj):
    acc = lambda shape: pl.BlockSpec((1,) + shape,
                                     lambda b, i: tuple([b] + [0] * len(shape)))
    return pl.pallas_call(
        _stats_body,
        grid=(2, _NB),
        in_specs=[
            _row_spec(_HID),
            _row_spec(_KG),
            _wspec((_HID, _HID)),
            _wspec((1, _HID)),
            _wspec((_HID, _HID)),
            _wspec((1, _HID)),
            _wspec((_MFEAT, _HID)),
        ],
        out_specs=[acc((_KG, _MFEAT, _HID)), acc((_KG, _HID)),
                   acc((_KG, _MFEAT)), acc((1, _KG)), acc((1, 1))],
        out_shape=[
            jax.ShapeDtypeStruct((2, _KG, _MFEAT, _HID), jnp.float32),
            jax.ShapeDtypeStruct((2, _KG, _HID), jnp.float32),
            jax.ShapeDtypeStruct((2, _KG, _MFEAT), jnp.float32),
            jax.ShapeDtypeStruct((2, 1, _KG), jnp.float32),
            jax.ShapeDtypeStruct((2, 1, 1), jnp.float32),
        ],
    )(z, eg, wk, bk, wv, bv, pj)


def _apply_body(z_ref, wq_ref, bq_ref, pj_ref, s1_ref, t_ref, s1s_ref, ts_ref,
                gm_ref, wo_ref, bo_ref, lg_ref, lb_ref, o_ref):
    z = z_ref[...]
    zq = (jnp.dot(z, wq_ref[0], preferred_element_type=jnp.float32)
          + bq_ref[0, 0]) * _DN
    dd = lax.dot_general(zq, pj_ref[0], (((1,), (1,)), ((), ())),
                         preferred_element_type=jnp.float32)
    diag = 0.5 * jnp.sum(zq * zq, -1, keepdims=True)
    rmax = jnp.max(dd, -1, keepdims=True)
    qp = jnp.exp(dd - diag - rmax) + 1e-6
    em = jnp.exp(-gm_ref[0, 0, 0])
    acc = jnp.zeros((z.shape[0], _HID), jnp.float32)
    for k in range(_KG):
        kvsk = em * s1_ref[0, k] + 1e-6 * t_ref[0, k][None, :]
        kssk = em * s1s_ref[0, k] + 1e-6 * ts_ref[0, 0, k]
        num = jnp.dot(qp, kvsk, preferred_element_type=jnp.float32)
        den = jnp.sum(qp * kssk[None, :], -1, keepdims=True)
        acc = acc + num / den
    z_att = acc * (1.0 / _KG)
    out = jnp.dot(z_att, wo_ref[0], preferred_element_type=jnp.float32)
    out = out + bo_ref[0, 0] + z
    o_ref[...] = _ln_elu(out, lg_ref[0, 0], lb_ref[0, 0])


def _apply(z, wq, bq, pj, s1, t, s1s, ts, gm, wo, bo, lg, lb):
    acc = lambda shape: pl.BlockSpec((1,) + shape,
                                     lambda b, i: tuple([b] + [0] * len(shape)))
    return pl.pallas_call(
        _apply_body,
        grid=(2, _NB),
        in_specs=[
            _row_spec(_HID),
            _wspec((_HID, _HID)), _wspec((1, _HID)), _wspec((_MFEAT, _HID)),
            acc((_KG, _MFEAT, _HID)), acc((_KG, _HID)), acc((_KG, _MFEAT)),
            acc((1, _KG)), acc((1, 1)),
            _wspec((_HID, _HID)), _wspec((1, _HID)),
            _wspec((1, _HID)), _wspec((1, _HID)),
        ],
        out_specs=_row_spec(_HID),
        out_shape=jax.ShapeDtypeStruct((2 * _N, _HID), jnp.float32),
    )(z, wq, bq, pj, s1, t, s1s, ts, gm, wo, bo, lg, lb)


def _epilogue_body(z0_ref, z1_ref, z2_ref, z3_ref, w_ref, b_ref, o_ref):
    w = w_ref[0]
    out = jnp.dot(z0_ref[...], w[0:128], preferred_element_type=jnp.float32)
    out += jnp.dot(z1_ref[...], w[128:256], preferred_element_type=jnp.float32)
    out += jnp.dot(z2_ref[...], w[256:384], preferred_element_type=jnp.float32)
    out += jnp.dot(z3_ref[...], w[384:512], preferred_element_type=jnp.float32)
    o_ref[...] = out + b_ref[0, 0]


def _epilogue(z0, z1, z2, z3, w, b):
    return pl.pallas_call(
        _epilogue_body,
        grid=(2, _NB),
        in_specs=[_row_spec(_HID)] * 4 + [_wspec((4 * _HID, _HID)),
                                          _wspec((1, _HID))],
        out_specs=_row_spec(_HID),
        out_shape=jax.ShapeDtypeStruct((2 * _N, _HID), jnp.float32),
    )(z0, z1, z2, z3, w, b)


def _deg_of(degp_blk):
    return degp_blk[0, :, 0] + degp_blk[1, :, 0] + 1.0


_PSPEC = pl.BlockSpec((1, _BLK, _HID), lambda p, i: (p, i, 0))
_DEGSPEC2 = pl.BlockSpec((2, _BLK, 16), lambda p, i: (0, i, 0))
_DEGSPEC1 = pl.BlockSpec((2, _BLK, 16), lambda i: (0, i, 0))


def _gcn_pre_body(e_ref, degp_ref, w_ref, o_ref):
    dinv = lax.rsqrt(_deg_of(degp_ref[...]))
    xw = jnp.dot(e_ref[...], w_ref[0], preferred_element_type=jnp.float32)
    o_ref[0] = dinv[:, None] * xw


def _gcn_pre(e, degp, w):
    return pl.pallas_call(
        _gcn_pre_body,
        grid=(2, _NB),
        in_specs=[_row_spec(_HID), _DEGSPEC2, _wspec((_HID, _HID))],
        out_specs=_PSPEC,
        out_shape=jax.ShapeDtypeStruct((2, _N, _HID), jnp.float32),
    )(e, degp, w)


def _gcn_mid_body(acc_ref, xs_ref, degp_ref, b_ref, a_ref, w2_ref, o_ref):
    dinv = lax.rsqrt(_deg_of(degp_ref[...]))
    h = dinv[:, None] * (acc_ref[0] + xs_ref[0]) + b_ref[0, 0]
    h = jnp.where(h >= 0, h, a_ref[0, 0] * h)
    o_ref[0] = dinv[:, None] * jnp.dot(h, w2_ref[0],
                                       preferred_element_type=jnp.float32)


def _gcn_mid(acc, xs, degp, b, a, w2):
    return pl.pallas_call(
        _gcn_mid_body,
        grid=(2, _NB),
        in_specs=[_PSPEC, _PSPEC, _DEGSPEC2, _wspec((1, _HID)), _wspec((1, _HID)),
                  _wspec((_HID, _HID))],
        out_specs=_PSPEC,
        out_shape=jax.ShapeDtypeStruct((2, _N, _HID), jnp.float32),
    )(acc, xs, degp, b, a, w2)


def _combine_body(acc_ref, xs_ref, degp_ref, b_ref, a_ref, al_ref, o_ref):
    dinv = lax.rsqrt(_deg_of(degp_ref[...]))
    gs = []
    for p in (0, 1):
        g = dinv[:, None] * (acc_ref[p] + xs_ref[p]) + b_ref[...][p]
        g = jnp.where(g >= 0, g, a_ref[...][p] * g)
        nrm = jnp.sqrt(jnp.sum(g * g, -1, keepdims=True))
        gs.append(g / jnp.maximum(nrm, 1e-12))
    a0 = al_ref[0, 0]
    a1 = al_ref[0, 1]
    m = jnp.maximum(a0, a1)
    e0 = jnp.exp(a0 - m)
    e1 = jnp.exp(a1 - m)
    o_ref[...] = (e0 / (e0 + e1)) * gs[0] + (e1 / (e0 + e1)) * gs[1]


def _combine(acc2, xs2, degp, b2, a2, alpha):
    full2 = pl.BlockSpec((2, _BLK, _HID), lambda i: (0, i, 0))
    return pl.pallas_call(
        _combine_body,
        grid=(_NB,),
        in_specs=[full2, full2, _DEGSPEC1,
                  pl.BlockSpec((2, _HID), lambda i: (0, 0)),
                  pl.BlockSpec((2, _HID), lambda i: (0, 0)),
                  pl.BlockSpec((1, 2), lambda i: (0, 0))],
        out_specs=pl.BlockSpec((_BLK, _HID), lambda i: (i, 0)),
        out_shape=jax.ShapeDtypeStruct((_N, _HID), jnp.float32),
    )(acc2, xs2, degp, b2, a2, alpha)


def _dec_body(z_ref, w_ref, o_ref):
    o_ref[0] = jnp.dot(z_ref[...], w_ref[0], preferred_element_type=jnp.float32)


def _dec_mm(z, w3):
    return pl.pallas_call(
        _dec_body,
        grid=(3, _NB),
        in_specs=[pl.BlockSpec((_BLK, _HID), lambda t, i: (i, 0)),
                  pl.BlockSpec((1, _HID, _HID), lambda t, i: (t, 0, 0))],
        out_specs=pl.BlockSpec((1, _BLK, _HID), lambda t, i: (t, i, 0)),
        out_shape=jax.ShapeDtypeStruct((3, _N, _HID), jnp.float32),
    )(z, w3)


def _addp_body(p_ref, o_ref):
    o_ref[...] = p_ref[0] + p_ref[1]


def _add_partials(parts):
    return pl.pallas_call(
        _addp_body,
        grid=(_NB,),
        in_specs=[pl.BlockSpec((2, _BLK, _HID), lambda i: (0, i, 0))],
        out_specs=pl.BlockSpec((_BLK, _HID), lambda i: (i, 0)),
        out_shape=jax.ShapeDtypeStruct((_N, _HID), jnp.float32),
    )(parts)


_CH = 80
_NROW_T = 632
_NPAD = 16 * _NROW_T
_ZR = _NROW_T


def _zero_rows(zb, d):
    zeros16 = jnp.zeros((16,), jnp.float32)

    def zrow(r, c):
        for cc in range(d // 16):
            zb[r, pl.ds(cc * 16, 16)] = zeros16
        return c

    lax.fori_loop(0, _ZR, zrow, 0)


def _zero_acc_and_barrier(zb, acc, row0):
    pltpu.sync_copy(zb, acc.at[pl.ds(row0, _ZR)])
    plsc.subcore_barrier()


def _make_scatter(d, tpc, scaled=True):
    pt = _E // 16
    nch = pt // _CH

    @functools.partial(
        pl.kernel,
        out_type=jax.ShapeDtypeStruct((2 * tpc, _NPAD, d), jnp.float32),
        mesh=plsc.VectorSubcoreMesh(core_axis_name="c", subcore_axis_name="s"),
        compiler_params=pltpu.CompilerParams(use_tc_tiling_on_sc=False),
        scratch_types=[
            pltpu.VMEM((nch, _CH), jnp.int32),
            pltpu.VMEM((nch, _CH), jnp.int32),
            pltpu.VMEM((nch, _CH), jnp.float32),
            pltpu.VMEM((2, _CH, d), jnp.float32),
            pltpu.VMEM((_ZR, d), jnp.float32),
            pltpu.VMEM_SHARED((_NPAD, d), jnp.float32),
            pltpu.SemaphoreType.DMA,
            pltpu.SemaphoreType.DMA,
        ],
    )
    def k(*args):
        tbls = args[:2 * tpc]
        (gi_h, si_h, val_h, out_h,
         gi_v, si_v, val_v, rows_v, zb, acc, sem0, sem1) = args[2 * tpc:]
        sems = (sem0, sem1)
        cid = lax.axis_index("c")
        sid = lax.axis_index("s")
        row0 = sid * _NROW_T
        _zero_rows(zb, d)
        pltpu.sync_copy(gi_h.at[pl.ds(sid * nch, nch)], gi_v)
        pltpu.sync_copy(si_h.at[pl.ds(sid * nch, nch)], si_v)
        pltpu.sync_copy(val_h.at[pl.ds(sid * nch, nch)], val_v)

        def run(tbl, tglob):
            pltpu.sync_copy(zb, acc.at[pl.ds(row0, _ZR)])
            plsc.subcore_barrier()

            pltpu.async_copy(tbl.at[gi_v.at[0]], rows_v.at[0], sems[0])

            def step(jj, b, issue_next):
                if issue_next:
                    pltpu.async_copy(tbl.at[gi_v.at[jj + 1]],
                                     rows_v.at[1 - b], sems[1 - b])
                pltpu.make_async_copy(tbl.at[gi_v.at[jj]],
                                      rows_v.at[b], sems[b]).wait()
                if scaled:
                    def sgrp(g, c2, b=b, jj=jj):
                        v16 = val_v[jj, pl.ds(g * 16, 16)]
                        for r in range(16):
                            s = v16[r]
                            for cc in range(d // 16):
                                sl = pl.ds(cc * 16, 16)
                                rows_v[b, g * 16 + r, sl] = \
                                    rows_v[b, g * 16 + r, sl] * s
                        return c2

                    lax.fori_loop(0, _CH // 16, sgrp, 0)
                pltpu.sync_copy(rows_v.at[b], acc.at[si_v.at[jj]], add=True)

            def body(jh, c):
                for b in range(2):
                    step(jh * 2 + b, b, True)
                return c

            lax.fori_loop(0, nch // 2 - 1, body, 0)
            step(nch - 2, 0, True)
            step(nch - 1, 1, False)
            plsc.subcore_barrier()
            pltpu.sync_copy(acc.at[pl.ds(row0, _ZR)],
                            out_h.at[tglob, pl.ds(row0, _ZR)])

        for tloc in range(tpc):
            @pl.when(cid == 0)
            def _(tloc=tloc):
                run(tbls[tloc], tloc)

            @pl.when(cid == 1)
            def _(tloc=tloc):
                run(tbls[tpc + tloc], tpc + tloc)

    return k


def _make_deg_kernel():
    @functools.partial(
        pl.kernel,
        out_type=jax.ShapeDtypeStruct((2, _NPAD, 16), jnp.float32),
        mesh=plsc.VectorSubcoreMesh(core_axis_name="c", subcore_axis_name="s"),
        compiler_params=pltpu.CompilerParams(use_tc_tiling_on_sc=False),
        scratch_types=[
            pltpu.VMEM((_CH,), jnp.int32),
            pltpu.VMEM((_CH, 16), jnp.float32),
            pltpu.VMEM((_ZR, 16), jnp.float32),
            pltpu.VMEM_SHARED((_NPAD, 16), jnp.float32),
        ],
    )
    def _deg_kernel(dst_h, out_h, dst_v, ones_v, zb, acc):
        cid = lax.axis_index("c")
        sid = lax.axis_index("s")
        row0 = sid * _NROW_T
        ones16 = jnp.ones((16,), jnp.float32)

        def orow(r, c):
            ones_v[r, pl.ds(0, 16)] = ones16
            return c

        lax.fori_loop(0, _CH, orow, 0)
        _zero_rows(zb, 16)
        _zero_acc_and_barrier(zb, acc, row0)

        pt = _E // 32
        nch = pt // _CH

        def body(j, c):
            base = (cid * 16 + sid) * pt + j * _CH
            pltpu.sync_copy(dst_h.at[pl.ds(base, _CH)], dst_v)
            pltpu.sync_copy(ones_v, acc.at[dst_v], add=True)
            return c

        lax.fori_loop(0, nch, body, 0)
        plsc.subcore_barrier()
        pltpu.sync_copy(acc.at[pl.ds(row0, _ZR)],
                        out_h.at[cid, pl.ds(row0, _ZR)])

    return _deg_kernel


_SC_CACHE = {}


def _get_sc(name):
    if name not in _SC_CACHE:
        _SC_CACHE['deg'] = _make_deg_kernel()
        _SC_CACHE['conv'] = _make_scatter(32, tpc=4, scaled=False)
        _SC_CACHE['dec'] = _make_scatter(32, tpc=6)
    return _SC_CACHE[name]


def _stk1(arrs):
    return jnp.stack(arrs)[:, None, :]

def _encode(x, p, projs, egs):
    w0 = jnp.stack([jnp.pad(pe['fc0_w'], ((0, _IN1 - pe['fc0_w'].shape[0]),
                                          (0, 0))) for pe in p])
    b0 = _stk1([pe['fc0_b'] for pe in p])
    g0 = _stk1([pe['ln0_g'] for pe in p])
    bb0 = _stk1([pe['ln0_b'] for pe in p])
    z = _prologue(x, w0, b0, g0, bb0)
    layers = [z]
    for i in range(_NLAYERS):
        cs = [pe['conv%d' % i] for pe in p]
        pj = jnp.stack([projs[0][i], projs[1][i]])
        eg = jnp.concatenate([egs[0][i], egs[1][i]], axis=0)
        wk = jnp.stack([c['Wk_w'] for c in cs])
        bk = _stk1([c['Wk_b'] for c in cs])
        wv = jnp.stack([c['Wv_w'] for c in cs])
        bv = _stk1([c['Wv_b'] for c in cs])
        s1, t, s1s, ts, gm = _stats(z, eg, wk, bk, wv, bv, pj)
        wq = jnp.stack([c['Wq_w'] for c in cs])
        bq = _stk1([c['Wq_b'] for c in cs])
        wo = jnp.stack([c['Wo_w'] for c in cs])
        bo = _stk1([c['Wo_b'] for c in cs])
        lg = _stk1([pe['ln%d_g' % (i + 1)] for pe in p])
        lb = _stk1([pe['ln%d_b' % (i + 1)] for pe in p])
        z = _apply(z, wq, bq, pj, s1, t, s1s, ts, gm, wo, bo, lg, lb)
        layers.append(z)
    w1 = jnp.stack([pe['fc1_w'] for pe in p])
    b1 = _stk1([pe['fc1_b'] for pe in p])
    return _epilogue(layers[0], layers[1], layers[2], layers[3], w1, b1)


def kernel(x1, x2, edge_index, adj_values, params):
    projs_all, egs_all = _rand_consts()
    projs = (projs_all[:_NLAYERS], projs_all[_NLAYERS:])
    egs = (egs_all[:_NLAYERS], egs_all[_NLAYERS:])

    src = edge_index[0]
    dst = edge_index[1]
    src2 = src.reshape(_E // _CH, _CH)
    dst2 = dst.reshape(_E // _CH, _CH)
    adj2 = adj_values.reshape(_E // _CH, _CH)

    degp = _get_sc('deg')(dst)

    x = jnp.concatenate(
        [x1, jnp.pad(x2, ((0, 0), (0, _IN1 - _IN2)))], axis=0)
    e = _encode(x, (params['enc1'], params['enc2']), projs, egs)

    f = params['fus']
    sc_conv = _get_sc('conv')
    sc_dec = _get_sc('dec')
    ones2 = jnp.ones((_E // _CH, _CH), jnp.float32)

    def split32(a):
        return a.reshape(2, _N, 4, 32).transpose(0, 2, 1, 3).reshape(8, _N, 32)

    def join32(a8):
        return a8.reshape(2, 4, _NPAD, 32).transpose(0, 2, 1, 3).reshape(
            2, _NPAD, 128)

    w12 = jnp.stack([f['c1_w'], f['c2_w']])
    xs = _gcn_pre(e, degp, w12)
    xs8 = split32(xs)
    acc1 = join32(sc_conv(*(xs8[i] for i in range(8)), src2, dst2, ones2))

    b12 = _stk1([f['c1_b'], f['c2_b']])
    a13 = _stk1([f['prelu1'], f['prelu3']])
    w34 = jnp.stack([f['c3_w'], f['c4_w']])
    xs2 = _gcn_mid(acc1, xs, degp, b12, a13, w34)
    xs28 = split32(xs2)
    acc2 = join32(sc_conv(*(xs28[i] for i in range(8)), src2, dst2, ones2))

    b34 = jnp.stack([f['c3_b'], f['c4_b']])
    a24 = jnp.stack([f['prelu2'], f['prelu4']])
    z = _combine(acc2, xs2, degp, b34, a24, f['alpha'].reshape(1, 2))

    w3 = jnp.stack([params['dec1_w'][:, :128], params['dec1_w'][:, 128:],
                    params['dec2_w']])
    zd = _dec_mm(z, w3)
    zd12 = zd.reshape(3, _N, 4, 32).transpose(0, 2, 1, 3).reshape(12, _N, 32)

    accd = sc_dec(*(zd12[i] for i in range(12)), dst2, src2, adj2)
    r1 = jnp.concatenate([accd[i, :_N] for i in range(8)], axis=1)
    r2 = jnp.concatenate([accd[i, :_N] for i in range(8, 12)], axis=1)

    return (z, r1, r2)


# speedup vs baseline: 1.4990x; 1.0365x over previous
import functools
import math

import jax
import jax.numpy as jnp
import numpy as np
from jax import lax
from jax.experimental import pallas as pl
from jax.experimental.pallas import tpu as pltpu
from jax.experimental.pallas import tpu_sc as plsc

_N = 10000
_E = 320000
_IN1, _OUT1 = 256, 128
_IN2, _OUT2 = 128, 128
_HID = 128
_NLAYERS = 3
_MFEAT = 30
_KG = 10
_DN = 1.0 / math.sqrt(math.sqrt(128.0))

_BLK = 2000
_NB = _N // _BLK


def _rand_consts_raw():
    rk = jax.random.key(42)
    projs, egs = [], []
    for enc in (1, 2):
        key = jax.random.fold_in(rk, enc)
        for i in range(_NLAYERS):
            lk = jax.random.fold_in(key, i)
            kp, kg = jax.random.split(lk)
            k1, k2 = jax.random.split(kp)
            q, _ = jnp.linalg.qr(jax.random.normal(k1, (_HID, _HID)))
            mult = jnp.linalg.norm(jax.random.normal(k2, (_MFEAT, _HID)), axis=1)
            proj = mult[:, None] * q[:_MFEAT]
            g = -jnp.log(jax.random.exponential(kg, (_N, 1, _KG)))
            projs.append(proj.astype(jnp.float32))
            egs.append(jnp.exp(g).reshape(_N, _KG).astype(jnp.float32))
    return projs, egs


@functools.lru_cache(maxsize=None)
def _rand_consts_host():
    with jax.ensure_compile_time_eval(), \
         jax.default_device(jax.local_devices(backend="cpu")[0]):
        projs, egs = _rand_consts_raw()
        return ([np.asarray(p) for p in projs], [np.asarray(e) for e in egs])


def _rand_consts():
    try:
        return _rand_consts_host()
    except Exception:
        return _rand_consts_raw()


def _ln_elu(z, g, b):
    mu = jnp.mean(z, -1, keepdims=True)
    zc = z - mu
    var = jnp.mean(zc * zc, -1, keepdims=True)
    z = zc * lax.rsqrt(var + 1e-5) * g + b
    return jnp.where(z > 0, z, jnp.exp(jnp.minimum(z, 0.0)) - 1.0)


def _row_spec(width):
    return pl.BlockSpec((_BLK, width), lambda b, i: (b * _NB + i, 0))


def _wspec(shape):
    nz = len(shape)
    return pl.BlockSpec((1,) + shape, lambda b, i, nz=nz: (b,) + (0,) * nz)


def _prologue_body(x_ref, w_ref, b_ref, g_ref, bb_ref, o_ref):
    z = jnp.dot(x_ref[...], w_ref[0], preferred_element_type=jnp.float32)
    z = z + b_ref[0, 0]
    o_ref[...] = _ln_elu(z, g_ref[0, 0], bb_ref[0, 0])


def _prologue(x, w, b, g, bb):
    return pl.pallas_call(
        _prologue_body,
        grid=(2, _NB),
        in_specs=[
            _row_spec(x.shape[1]),
            _wspec(w.shape[1:]),
            _wspec((1, _HID)),
            _wspec((1, _HID)),
            _wspec((1, _HID)),
        ],
        out_specs=_row_spec(_HID),
        out_shape=jax.ShapeDtypeStruct((2 * _N, _HID), jnp.float32),
    )(x, w, b, g, bb)


def _stats_body(z_ref, eg_ref, wk_ref, bk_ref, wv_ref, bv_ref, pj_ref,
                s1_ref, t_ref, s1s_ref, ts_ref, gm_ref):
    i = pl.program_id(1)

    @pl.when(i == 0)
    def _():
        s1_ref[...] = jnp.zeros_like(s1_ref)
        t_ref[...] = jnp.zeros_like(t_ref)
        s1s_ref[...] = jnp.zeros_like(s1s_ref)
        ts_ref[...] = jnp.zeros_like(ts_ref)
        gm_ref[...] = jnp.full(gm_ref.shape, -jnp.inf, jnp.float32)

    z = z_ref[...]
    zk = (jnp.dot(z, wk_ref[0], preferred_element_type=jnp.float32)
          + bk_ref[0, 0]) * _DN
    dd = lax.dot_general(zk, pj_ref[0], (((1,), (1,)), ((), ())),
                         preferred_element_type=jnp.float32)
    diag = 0.5 * jnp.sum(zk * zk, -1, keepdims=True)
    p = jnp.exp(dd - diag)
    v = jnp.dot(z, wv_ref[0], preferred_element_type=jnp.float32) + bv_ref[0, 0]
    eg = eg_ref[...]
    for k in range(_KG):
        a = p * eg[:, k:k + 1]
        s1k = lax.dot_general(a, v, (((0,), (0,)), ((), ())),
                              preferred_element_type=jnp.float32)
        s1_ref[0, k] += s1k
    t_ref[0] += lax.dot_general(eg, v, (((0,), (0,)), ((), ())),
                                preferred_element_type=jnp.float32)
    s1s_ref[0] += lax.dot_general(eg, p, (((0,), (0,)), ((), ())),
                                  preferred_element_type=jnp.float32)
    ts_ref[...] += jnp.sum(eg, axis=0).reshape(1, 1, _KG)
    gm_ref[...] = jnp.maximum(gm_ref[...], jnp.reshape(jnp.max(dd), (1, 1, 1)))


def _stats(z, eg, wk, bk, wv, bv, pj):
    acc = lambda shape: pl.BlockSpec((1,) + shape,
                                     lambda b, i: tuple([b] + [0] * len(shape)))
    return pl.pallas_call(
        _stats_body,
        grid=(2, _NB),
        in_specs=[
            _row_spec(_HID),
            _row_spec(_KG),
            _wspec((_HID, _HID)),
            _wspec((1, _HID)),
            _wspec((_HID, _HID)),
            _wspec((1, _HID)),
            _wspec((_MFEAT, _HID)),
        ],
        out_specs=[acc((_KG, _MFEAT, _HID)), acc((_KG, _HID)),
                   acc((_KG, _MFEAT)), acc((1, _KG)), acc((1, 1))],
        out_shape=[
            jax.ShapeDtypeStruct((2, _KG, _MFEAT, _HID), jnp.float32),
            jax.ShapeDtypeStruct((2, _KG, _HID), jnp.float32),
            jax.ShapeDtypeStruct((2, _KG, _MFEAT), jnp.float32),
            jax.ShapeDtypeStruct((2, 1, _KG), jnp.float32),
            jax.ShapeDtypeStruct((2, 1, 1), jnp.float32),
        ],
    )(z, eg, wk, bk, wv, bv, pj)


def _apply_body(z_ref, wq_ref, bq_ref, pj_ref, s1_ref, t_ref, s1s_ref, ts_ref,
                gm_ref, wo_ref, bo_ref, lg_ref, lb_ref, o_ref):
    z = z_ref[...]
    zq = (jnp.dot(z, wq_ref[0], preferred_element_type=jnp.float32)
          + bq_ref[0, 0]) * _DN
    dd = lax.dot_general(zq, pj_ref[0], (((1,), (1,)), ((), ())),
                         preferred_element_type=jnp.float32)
    diag = 0.5 * jnp.sum(zq * zq, -1, keepdims=True)
    rmax = jnp.max(dd, -1, keepdims=True)
    qp = jnp.exp(dd - diag - rmax) + 1e-6
    em = jnp.exp(-gm_ref[0, 0, 0])
    acc = jnp.zeros((z.shape[0], _HID), jnp.float32)
    for k in range(_KG):
        kvsk = em * s1_ref[0, k] + 1e-6 * t_ref[0, k][None, :]
        kssk = em * s1s_ref[0, k] + 1e-6 * ts_ref[0, 0, k]
        num = jnp.dot(qp, kvsk, preferred_element_type=jnp.float32)
        den = jnp.sum(qp * kssk[None, :], -1, keepdims=True)
        acc = acc + num / den
    z_att = acc * (1.0 / _KG)
    out = jnp.dot(z_att, wo_ref[0], preferred_element_type=jnp.float32)
    out = out + bo_ref[0, 0] + z
    o_ref[...] = _ln_elu(out, lg_ref[0, 0], lb_ref[0, 0])


def _apply(z, wq, bq, pj, s1, t, s1s, ts, gm, wo, bo, lg, lb):
    acc = lambda shape: pl.BlockSpec((1,) + shape,
                                     lambda b, i: tuple([b] + [0] * len(shape)))
    return pl.pallas_call(
        _apply_body,
        grid=(2, _NB),
        in_specs=[
            _row_spec(_HID),
            _wspec((_HID, _HID)), _wspec((1, _HID)), _wspec((_MFEAT, _HID)),
            acc((_KG, _MFEAT, _HID)), acc((_KG, _HID)), acc((_KG, _MFEAT)),
            acc((1, _KG)), acc((1, 1)),
            _wspec((_HID, _HID)), _wspec((1, _HID)),
            _wspec((1, _HID)), _wspec((1, _HID)),
        ],
        out_specs=_row_spec(_HID),
        out_shape=jax.ShapeDtypeStruct((2 * _N, _HID), jnp.float32),
    )(z, wq, bq, pj, s1, t, s1s, ts, gm, wo, bo, lg, lb)


def _deg_of(degp_blk):
    return degp_blk[0, :, 0] + degp_blk[1, :, 0] + 1.0


_PSPEC = pl.BlockSpec((1, _BLK, _HID), lambda p, i: (p, i, 0))
_DEGSPEC2 = pl.BlockSpec((2, _BLK, 16), lambda p, i: (0, i, 0))
_DEGSPEC1 = pl.BlockSpec((2, _BLK, 16), lambda i: (0, i, 0))


def _epi_pre_body(z0_ref, z1_ref, z2_ref, z3_ref, w_ref, b_ref, degp_ref,
                  wc_ref, o_ref):
    w = w_ref[0]
    out = jnp.dot(z0_ref[...], w[0:128], preferred_element_type=jnp.float32)
    out += jnp.dot(z1_ref[...], w[128:256], preferred_element_type=jnp.float32)
    out += jnp.dot(z2_ref[...], w[256:384], preferred_element_type=jnp.float32)
    out += jnp.dot(z3_ref[...], w[384:512], preferred_element_type=jnp.float32)
    e = out + b_ref[0, 0]
    dinv = lax.rsqrt(_deg_of(degp_ref[...]))
    o_ref[0] = dinv[:, None] * jnp.dot(e, wc_ref[0],
                                       preferred_element_type=jnp.float32)


def _epi_pre(z0, z1, z2, z3, w, b, degp, wc):
    return pl.pallas_call(
        _epi_pre_body,
        grid=(2, _NB),
        in_specs=[_row_spec(_HID)] * 4 + [_wspec((4 * _HID, _HID)),
                                          _wspec((1, _HID)), _DEGSPEC2,
                                          _wspec((_HID, _HID))],
        out_specs=_PSPEC,
        out_shape=jax.ShapeDtypeStruct((2, _N, _HID), jnp.float32),
    )(z0, z1, z2, z3, w, b, degp, wc)


def _gcn_mid_body(acc_ref, xs_ref, degp_ref, b_ref, a_ref, w2_ref, o_ref):
    dinv = lax.rsqrt(_deg_of(degp_ref[...]))
    h = dinv[:, None] * (acc_ref[0] + xs_ref[0]) + b_ref[0, 0]
    h = jnp.where(h >= 0, h, a_ref[0, 0] * h)
    o_ref[0] = dinv[:, None] * jnp.dot(h, w2_ref[0],
                                       preferred_element_type=jnp.float32)


def _gcn_mid(acc, xs, degp, b, a, w2):
    return pl.pallas_call(
        _gcn_mid_body,
        grid=(2, _NB),
        in_specs=[_PSPEC, _PSPEC, _DEGSPEC2, _wspec((1, _HID)), _wspec((1, _HID)),
                  _wspec((_HID, _HID))],
        out_specs=_PSPEC,
        out_shape=jax.ShapeDtypeStruct((2, _N, _HID), jnp.float32),
    )(acc, xs, degp, b, a, w2)


def _combine_body(acc_ref, xs_ref, degp_ref, b_ref, a_ref, al_ref, w3_ref,
                  o_ref, o2_ref):
    dinv = lax.rsqrt(_deg_of(degp_ref[...]))
    gs = []
    for p in (0, 1):
        g = dinv[:, None] * (acc_ref[p] + xs_ref[p]) + b_ref[...][p]
        g = jnp.where(g >= 0, g, a_ref[...][p] * g)
        nrm = jnp.sqrt(jnp.sum(g * g, -1, keepdims=True))
        gs.append(g / jnp.maximum(nrm, 1e-12))
    a0 = al_ref[0, 0]
    a1 = al_ref[0, 1]
    m = jnp.maximum(a0, a1)
    e0 = jnp.exp(a0 - m)
    e1 = jnp.exp(a1 - m)
    z = (e0 / (e0 + e1)) * gs[0] + (e1 / (e0 + e1)) * gs[1]
    o_ref[...] = z
    for t in range(3):
        zdt = jnp.dot(z, w3_ref[t], preferred_element_type=jnp.float32)
        for c in range(4):
            o2_ref[t * 4 + c] = zdt[:, c * 32:(c + 1) * 32]


def _combine_dec(acc2, xs2, degp, b2, a2, alpha, w3):
    full2 = pl.BlockSpec((2, _BLK, _HID), lambda i: (0, i, 0))
    return pl.pallas_call(
        _combine_body,
        grid=(_NB,),
        in_specs=[full2, full2, _DEGSPEC1,
                  pl.BlockSpec((2, _HID), lambda i: (0, 0)),
                  pl.BlockSpec((2, _HID), lambda i: (0, 0)),
                  pl.BlockSpec((1, 2), lambda i: (0, 0)),
                  pl.BlockSpec((3, _HID, _HID), lambda i: (0, 0, 0))],
        out_specs=[pl.BlockSpec((_BLK, _HID), lambda i: (i, 0)),
                   pl.BlockSpec((12, _BLK, 32), lambda i: (0, i, 0))],
        out_shape=[jax.ShapeDtypeStruct((_N, _HID), jnp.float32),
                   jax.ShapeDtypeStruct((12, _N, 32), jnp.float32)],
    )(acc2, xs2, degp, b2, a2, alpha, w3)


def _addp_body(p_ref, o_ref):
    o_ref[...] = p_ref[0] + p_ref[1]


def _add_partials(parts):
    return pl.pallas_call(
        _addp_body,
        grid=(_NB,),
        in_specs=[pl.BlockSpec((2, _BLK, _HID), lambda i: (0, i, 0))],
        out_specs=pl.BlockSpec((_BLK, _HID), lambda i: (i, 0)),
        out_shape=jax.ShapeDtypeStruct((_N, _HID), jnp.float32),
    )(parts)


_CH = 80
_NROW_T = 632
_NPAD = 16 * _NROW_T
_ZR = _NROW_T


def _zero_rows(zb, d):
    zeros16 = jnp.zeros((16,), jnp.float32)

    def zrow(r, c):
        for cc in range(d // 16):
            zb[r, pl.ds(cc * 16, 16)] = zeros16
        return c

    lax.fori_loop(0, _ZR, zrow, 0)


def _zero_acc_and_barrier(zb, acc, row0):
    pltpu.sync_copy(zb, acc.at[pl.ds(row0, _ZR)])
    plsc.subcore_barrier()


def _make_scatter(d, tpc, scaled=True):
    pt = _E // 16
    nch = pt // _CH

    @functools.partial(
        pl.kernel,
        out_type=jax.ShapeDtypeStruct((2 * tpc, _NPAD, d), jnp.float32),
        mesh=plsc.VectorSubcoreMesh(core_axis_name="c", subcore_axis_name="s"),
        compiler_params=pltpu.CompilerParams(use_tc_tiling_on_sc=False),
        scratch_types=[
            pltpu.VMEM((nch, _CH), jnp.int32),
            pltpu.VMEM((nch, _CH), jnp.int32),
            pltpu.VMEM((nch, _CH), jnp.float32),
            pltpu.VMEM((2, _CH, d), jnp.float32),
            pltpu.VMEM((_ZR, d), jnp.float32),
            pltpu.VMEM_SHARED((_NPAD, d), jnp.float32),
            pltpu.SemaphoreType.DMA,
            pltpu.SemaphoreType.DMA,
        ],
    )
    def k(*args):
        tbls = args[:2 * tpc]
        (gi_h, si_h, val_h, out_h,
         gi_v, si_v, val_v, rows_v, zb, acc, sem0, sem1) = args[2 * tpc:]
        sems = (sem0, sem1)
        cid = lax.axis_index("c")
        sid = lax.axis_index("s")
        row0 = sid * _NROW_T
        _zero_rows(zb, d)
        pltpu.sync_copy(gi_h.at[pl.ds(sid * nch, nch)], gi_v)
        pltpu.sync_copy(si_h.at[pl.ds(sid * nch, nch)], si_v)
        pltpu.sync_copy(val_h.at[pl.ds(sid * nch, nch)], val_v)

        def run(tbl, tglob):
            pltpu.sync_copy(zb, acc.at[pl.ds(row0, _ZR)])
            plsc.subcore_barrier()

            pltpu.async_copy(tbl.at[gi_v.at[0]], rows_v.at[0], sems[0])

            def step(jj, b, issue_next):
                if issue_next:
                    pltpu.async_copy(tbl.at[gi_v.at[jj + 1]],
                                     rows_v.at[1 - b], sems[1 - b])
                pltpu.make_async_copy(tbl.at[gi_v.at[jj]],
                                      rows_v.at[b], sems[b]).wait()
                if scaled:
                    def sgrp(g, c2, b=b, jj=jj):
                        v16 = val_v[jj, pl.ds(g * 16, 16)]
                        for r in range(16):
                            s = v16[r]
                            for cc in range(d // 16):
                                sl = pl.ds(cc * 16, 16)
                                rows_v[b, g * 16 + r, sl] = \
                                    rows_v[b, g * 16 + r, sl] * s
                        return c2

                    lax.fori_loop(0, _CH // 16, sgrp, 0)
                pltpu.sync_copy(rows_v.at[b], acc.at[si_v.at[jj]], add=True)

            def body(jh, c):
                for b in range(2):
                    step(jh * 2 + b, b, True)
                return c

            lax.fori_loop(0, nch // 2 - 1, body, 0)
            step(nch - 2, 0, True)
            step(nch - 1, 1, False)
            plsc.subcore_barrier()
            pltpu.sync_copy(acc.at[pl.ds(row0, _ZR)],
                            out_h.at[tglob, pl.ds(row0, _ZR)])

        for tloc in range(tpc):
            @pl.when(cid == 0)
            def _(tloc=tloc):
                run(tbls[tloc], tloc)

            @pl.when(cid == 1)
            def _(tloc=tloc):
                run(tbls[tpc + tloc], tpc + tloc)

    return k


def _make_deg_kernel():
    @functools.partial(
        pl.kernel,
        out_type=jax.ShapeDtypeStruct((2, _NPAD, 16), jnp.float32),
        mesh=plsc.VectorSubcoreMesh(core_axis_name="c", subcore_axis_name="s"),
        compiler_params=pltpu.CompilerParams(use_tc_tiling_on_sc=False),
        scratch_types=[
            pltpu.VMEM((_CH,), jnp.int32),
            pltpu.VMEM((_CH, 16), jnp.float32),
            pltpu.VMEM((_ZR, 16), jnp.float32),
            pltpu.VMEM_SHARED((_NPAD, 16), jnp.float32),
        ],
    )
    def _deg_kernel(dst_h, out_h, dst_v, ones_v, zb, acc):
        cid = lax.axis_index("c")
        sid = lax.axis_index("s")
        row0 = sid * _NROW_T
        ones16 = jnp.ones((16,), jnp.float32)

        def orow(r, c):
            ones_v[r, pl.ds(0, 16)] = ones16
            return c

        lax.fori_loop(0, _CH, orow, 0)
        _zero_rows(zb, 16)
        _zero_acc_and_barrier(zb, acc, row0)

        pt = _E // 32
        nch = pt // _CH

        def body(j, c):
            base = (cid * 16 + sid) * pt + j * _CH
            pltpu.sync_copy(dst_h.at[pl.ds(base, _CH)], dst_v)
            pltpu.sync_copy(ones_v, acc.at[dst_v], add=True)
            return c

        lax.fori_loop(0, nch, body, 0)
        plsc.subcore_barrier()
        pltpu.sync_copy(acc.at[pl.ds(row0, _ZR)],
                        out_h.at[cid, pl.ds(row0, _ZR)])

    return _deg_kernel


_SC_CACHE = {}


def _get_sc(name):
    if name not in _SC_CACHE:
        _SC_CACHE['deg'] = _make_deg_kernel()
        _SC_CACHE['conv'] = _make_scatter(32, tpc=4, scaled=False)
        _SC_CACHE['dec'] = _make_scatter(32, tpc=6)
    return _SC_CACHE[name]


def _stk1(arrs):
    return jnp.stack(arrs)[:, None, :]

def _encode(x, p, projs, egs):
    w0 = jnp.stack([jnp.pad(pe['fc0_w'], ((0, _IN1 - pe['fc0_w'].shape[0]),
                                          (0, 0))) for pe in p])
    b0 = _stk1([pe['fc0_b'] for pe in p])
    g0 = _stk1([pe['ln0_g'] for pe in p])
    bb0 = _stk1([pe['ln0_b'] for pe in p])
    z = _prologue(x, w0, b0, g0, bb0)
    layers = [z]
    for i in range(_NLAYERS):
        cs = [pe['conv%d' % i] for pe in p]
        pj = jnp.stack([projs[0][i], projs[1][i]])
        eg = jnp.concatenate([egs[0][i], egs[1][i]], axis=0)
        wk = jnp.stack([c['Wk_w'] for c in cs])
        bk = _stk1([c['Wk_b'] for c in cs])
        wv = jnp.stack([c['Wv_w'] for c in cs])
        bv = _stk1([c['Wv_b'] for c in cs])
        s1, t, s1s, ts, gm = _stats(z, eg, wk, bk, wv, bv, pj)
        wq = jnp.stack([c['Wq_w'] for c in cs])
        bq = _stk1([c['Wq_b'] for c in cs])
        wo = jnp.stack([c['Wo_w'] for c in cs])
        bo = _stk1([c['Wo_b'] for c in cs])
        lg = _stk1([pe['ln%d_g' % (i + 1)] for pe in p])
        lb = _stk1([pe['ln%d_b' % (i + 1)] for pe in p])
        z = _apply(z, wq, bq, pj, s1, t, s1s, ts, gm, wo, bo, lg, lb)
        layers.append(z)
    return layers


def kernel(x1, x2, edge_index, adj_values, params):
    projs_all, egs_all = _rand_consts()
    projs = (projs_all[:_NLAYERS], projs_all[_NLAYERS:])
    egs = (egs_all[:_NLAYERS], egs_all[_NLAYERS:])

    src = edge_index[0]
    dst = edge_index[1]
    src2 = src.reshape(_E // _CH, _CH)
    dst2 = dst.reshape(_E // _CH, _CH)
    adj2 = adj_values.reshape(_E // _CH, _CH)

    degp = _get_sc('deg')(dst)

    x = jnp.concatenate(
        [x1, jnp.pad(x2, ((0, 0), (0, _IN1 - _IN2)))], axis=0)
    encs = (params['enc1'], params['enc2'])
    layers = _encode(x, encs, projs, egs)

    f = params['fus']
    sc_conv = _get_sc('conv')
    sc_dec = _get_sc('dec')
    ones2 = jnp.ones((_E // _CH, _CH), jnp.float32)

    def split32(a):
        return a.reshape(2, _N, 4, 32).transpose(0, 2, 1, 3).reshape(8, _N, 32)

    def join32(a8):
        return a8.reshape(2, 4, _NPAD, 32).transpose(0, 2, 1, 3).reshape(
            2, _NPAD, 128)

    w1 = jnp.stack([pe['fc1_w'] for pe in encs])
    b1 = _stk1([pe['fc1_b'] for pe in encs])
    w12 = jnp.stack([f['c1_w'], f['c2_w']])
    xs = _epi_pre(layers[0], layers[1], layers[2], layers[3], w1, b1,
                  degp, w12)
    xs8 = split32(xs)
    acc1 = join32(sc_conv(*(xs8[i] for i in range(8)), src2, dst2, ones2))

    b12 = _stk1([f['c1_b'], f['c2_b']])
    a13 = _stk1([f['prelu1'], f['prelu3']])
    w34 = jnp.stack([f['c3_w'], f['c4_w']])
    xs2 = _gcn_mid(acc1, xs, degp, b12, a13, w34)
    xs28 = split32(xs2)
    acc2 = join32(sc_conv(*(xs28[i] for i in range(8)), src2, dst2, ones2))

    b34 = jnp.stack([f['c3_b'], f['c4_b']])
    a24 = jnp.stack([f['prelu2'], f['prelu4']])
    w3 = jnp.stack([params['dec1_w'][:, :128], params['dec1_w'][:, 128:],
                    params['dec2_w']])
    z, zd12 = _combine_dec(acc2, xs2, degp, b34, a24,
                           f['alpha'].reshape(1, 2), w3)

    accd = sc_dec(*(zd12[i] for i in range(12)), dst2, src2, adj2)
    r1 = jnp.concatenate([accd[i, :_N] for i in range(8)], axis=1)
    r2 = jnp.concatenate([accd[i, :_N] for i in range(8, 12)], axis=1)

    return (z, r1, r2)


# speedup vs baseline: 1.4992x; 1.0002x over previous
import functools
import math

import jax
import jax.numpy as jnp
import numpy as np
from jax import lax
from jax.experimental import pallas as pl
from jax.experimental.pallas import tpu as pltpu
from jax.experimental.pallas import tpu_sc as plsc

_N = 10000
_E = 320000
_IN1, _OUT1 = 256, 128
_IN2, _OUT2 = 128, 128
_HID = 128
_NLAYERS = 3
_MFEAT = 30
_KG = 10
_DN = 1.0 / math.sqrt(math.sqrt(128.0))

_BLK = 2000
_NB = _N // _BLK


def _rand_consts_raw():
    rk = jax.random.key(42)
    projs, egs = [], []
    for enc in (1, 2):
        key = jax.random.fold_in(rk, enc)
        for i in range(_NLAYERS):
            lk = jax.random.fold_in(key, i)
            kp, kg = jax.random.split(lk)
            k1, k2 = jax.random.split(kp)
            q, _ = jnp.linalg.qr(jax.random.normal(k1, (_HID, _HID)))
            mult = jnp.linalg.norm(jax.random.normal(k2, (_MFEAT, _HID)), axis=1)
            proj = mult[:, None] * q[:_MFEAT]
            g = -jnp.log(jax.random.exponential(kg, (_N, 1, _KG)))
            projs.append(proj.astype(jnp.float32))
            egs.append(jnp.exp(g).reshape(_N, _KG).astype(jnp.float32))
    return projs, egs


@functools.lru_cache(maxsize=None)
def _rand_consts_host():
    with jax.ensure_compile_time_eval(), \
         jax.default_device(jax.local_devices(backend="cpu")[0]):
        projs, egs = _rand_consts_raw()
        return ([np.asarray(p) for p in projs], [np.asarray(e) for e in egs])


def _rand_consts():
    try:
        return _rand_consts_host()
    except Exception:
        return _rand_consts_raw()


def _ln_elu(z, g, b):
    mu = jnp.mean(z, -1, keepdims=True)
    zc = z - mu
    var = jnp.mean(zc * zc, -1, keepdims=True)
    z = zc * lax.rsqrt(var + 1e-5) * g + b
    return jnp.where(z > 0, z, jnp.exp(jnp.minimum(z, 0.0)) - 1.0)


def _row_spec(width):
    return pl.BlockSpec((_BLK, width), lambda b, i: (b * _NB + i, 0))


def _wspec(shape):
    nz = len(shape)
    return pl.BlockSpec((1,) + shape, lambda b, i, nz=nz: (b,) + (0,) * nz)


def _prologue_body(x_ref, w_ref, b_ref, g_ref, bb_ref, o_ref):
    z = jnp.dot(x_ref[...], w_ref[0], preferred_element_type=jnp.float32)
    z = z + b_ref[0, 0]
    o_ref[...] = _ln_elu(z, g_ref[0, 0], bb_ref[0, 0])


def _prologue(x, w, b, g, bb):
    return pl.pallas_call(
        _prologue_body,
        grid=(2, _NB),
        in_specs=[
            _row_spec(x.shape[1]),
            _wspec(w.shape[1:]),
            _wspec((1, _HID)),
            _wspec((1, _HID)),
            _wspec((1, _HID)),
        ],
        out_specs=_row_spec(_HID),
        out_shape=jax.ShapeDtypeStruct((2 * _N, _HID), jnp.float32),
    )(x, w, b, g, bb)


def _layer_body(z_ref, eg_ref, wk_ref, bk_ref, wv_ref, bv_ref, pj_ref,
                wq_ref, bq_ref, wo_ref, bo_ref, lg_ref, lb_ref, o_ref,
                s1_ref, t_ref, s1s_ref, ts_ref, gm_ref):
    ph = pl.program_id(1)
    i = pl.program_id(2)

    @pl.when((ph == 0) & (i == 0))
    def _():
        s1_ref[...] = jnp.zeros_like(s1_ref)
        t_ref[...] = jnp.zeros_like(t_ref)
        s1s_ref[...] = jnp.zeros_like(s1s_ref)
        ts_ref[...] = jnp.zeros_like(ts_ref)
        gm_ref[...] = jnp.full(gm_ref.shape, -jnp.inf, jnp.float32)

    z = z_ref[...]

    @pl.when(ph == 0)
    def _():
        zk = (jnp.dot(z, wk_ref[0], preferred_element_type=jnp.float32)
              + bk_ref[0, 0]) * _DN
        dd = lax.dot_general(zk, pj_ref[0], (((1,), (1,)), ((), ())),
                             preferred_element_type=jnp.float32)
        diag = 0.5 * jnp.sum(zk * zk, -1, keepdims=True)
        p = jnp.exp(dd - diag)
        v = (jnp.dot(z, wv_ref[0], preferred_element_type=jnp.float32)
             + bv_ref[0, 0])
        eg = eg_ref[...]
        for k in range(_KG):
            a = p * eg[:, k:k + 1]
            s1k = lax.dot_general(a, v, (((0,), (0,)), ((), ())),
                                  preferred_element_type=jnp.float32)
            s1_ref[k] += s1k
        t_ref[...] += lax.dot_general(eg, v, (((0,), (0,)), ((), ())),
                                      preferred_element_type=jnp.float32)
        s1s_ref[...] += lax.dot_general(eg, p, (((0,), (0,)), ((), ())),
                                        preferred_element_type=jnp.float32)
        ts_ref[...] += jnp.sum(eg, axis=0).reshape(1, _KG)
        gm_ref[...] = jnp.maximum(gm_ref[...],
                                  jnp.reshape(jnp.max(dd), (1, 1)))

    @pl.when(ph == 1)
    def _():
        zq = (jnp.dot(z, wq_ref[0], preferred_element_type=jnp.float32)
              + bq_ref[0, 0]) * _DN
        dd = lax.dot_general(zq, pj_ref[0], (((1,), (1,)), ((), ())),
                             preferred_element_type=jnp.float32)
        diag = 0.5 * jnp.sum(zq * zq, -1, keepdims=True)
        rmax = jnp.max(dd, -1, keepdims=True)
        qp = jnp.exp(dd - diag - rmax) + 1e-6
        em = jnp.exp(-gm_ref[0, 0])
        acc = jnp.zeros((z.shape[0], _HID), jnp.float32)
        for k in range(_KG):
            kvsk = em * s1_ref[k] + 1e-6 * t_ref[k][None, :]
            kssk = em * s1s_ref[k] + 1e-6 * ts_ref[0, k]
            num = jnp.dot(qp, kvsk, preferred_element_type=jnp.float32)
            den = jnp.sum(qp * kssk[None, :], -1, keepdims=True)
            acc = acc + num / den
        z_att = acc * (1.0 / _KG)
        out = jnp.dot(z_att, wo_ref[0], preferred_element_type=jnp.float32)
        out = out + bo_ref[0, 0] + z
        o_ref[...] = _ln_elu(out, lg_ref[0, 0], lb_ref[0, 0])


def _layer(z, eg, wk, bk, wv, bv, pj, wq, bq, wo, bo, lg, lb):
    rspec = lambda w: pl.BlockSpec((_BLK, w),
                                   lambda b, ph, i: (b * _NB + i, 0))
    wspec = lambda shape: pl.BlockSpec(
        (1,) + shape, lambda b, ph, i, nz=len(shape): (b,) + (0,) * nz)
    return pl.pallas_call(
        _layer_body,
        grid=(2, 2, _NB),
        in_specs=[
            rspec(_HID), rspec(_KG),
            wspec((_HID, _HID)), wspec((1, _HID)),
            wspec((_HID, _HID)), wspec((1, _HID)),
            wspec((_MFEAT, _HID)),
            wspec((_HID, _HID)), wspec((1, _HID)),
            wspec((_HID, _HID)), wspec((1, _HID)),
            wspec((1, _HID)), wspec((1, _HID)),
        ],
        out_specs=pl.BlockSpec((_BLK, _HID),
                               lambda b, ph, i: (b * _NB + i * ph, 0)),
        out_shape=jax.ShapeDtypeStruct((2 * _N, _HID), jnp.float32),
        scratch_shapes=[
            pltpu.VMEM((_KG, _MFEAT, _HID), jnp.float32),
            pltpu.VMEM((_KG, _HID), jnp.float32),
            pltpu.VMEM((_KG, _MFEAT), jnp.float32),
            pltpu.VMEM((1, _KG), jnp.float32),
            pltpu.VMEM((1, 1), jnp.float32),
        ],
    )(z, eg, wk, bk, wv, bv, pj, wq, bq, wo, bo, lg, lb)


def _deg_of(degp_blk):
    return degp_blk[0, :, 0] + degp_blk[1, :, 0] + 1.0


_PSPEC = pl.BlockSpec((1, _BLK, _HID), lambda p, i: (p, i, 0))
_DEGSPEC2 = pl.BlockSpec((2, _BLK, 16), lambda p, i: (0, i, 0))
_DEGSPEC1 = pl.BlockSpec((2, _BLK, 16), lambda i: (0, i, 0))


def _epi_pre_body(z0_ref, z1_ref, z2_ref, z3_ref, w_ref, b_ref, degp_ref,
                  wc_ref, o_ref):
    w = w_ref[0]
    out = jnp.dot(z0_ref[...], w[0:128], preferred_element_type=jnp.float32)
    out += jnp.dot(z1_ref[...], w[128:256], preferred_element_type=jnp.float32)
    out += jnp.dot(z2_ref[...], w[256:384], preferred_element_type=jnp.float32)
    out += jnp.dot(z3_ref[...], w[384:512], preferred_element_type=jnp.float32)
    e = out + b_ref[0, 0]
    dinv = lax.rsqrt(_deg_of(degp_ref[...]))
    o_ref[0] = dinv[:, None] * jnp.dot(e, wc_ref[0],
                                       preferred_element_type=jnp.float32)


def _epi_pre(z0, z1, z2, z3, w, b, degp, wc):
    return pl.pallas_call(
        _epi_pre_body,
        grid=(2, _NB),
        in_specs=[_row_spec(_HID)] * 4 + [_wspec((4 * _HID, _HID)),
                                          _wspec((1, _HID)), _DEGSPEC2,
                                          _wspec((_HID, _HID))],
        out_specs=_PSPEC,
        out_shape=jax.ShapeDtypeStruct((2, _N, _HID), jnp.float32),
    )(z0, z1, z2, z3, w, b, degp, wc)


def _gcn_mid_body(acc_ref, xs_ref, degp_ref, b_ref, a_ref, w2_ref, o_ref):
    dinv = lax.rsqrt(_deg_of(degp_ref[...]))
    h = dinv[:, None] * (acc_ref[0] + xs_ref[0]) + b_ref[0, 0]
    h = jnp.where(h >= 0, h, a_ref[0, 0] * h)
    o_ref[0] = dinv[:, None] * jnp.dot(h, w2_ref[0],
                                       preferred_element_type=jnp.float32)


def _gcn_mid(acc, xs, degp, b, a, w2):
    return pl.pallas_call(
        _gcn_mid_body,
        grid=(2, _NB),
        in_specs=[_PSPEC, _PSPEC, _DEGSPEC2, _wspec((1, _HID)), _wspec((1, _HID)),
                  _wspec((_HID, _HID))],
        out_specs=_PSPEC,
        out_shape=jax.ShapeDtypeStruct((2, _N, _HID), jnp.float32),
    )(acc, xs, degp, b, a, w2)


def _combine_body(acc_ref, xs_ref, degp_ref, b_ref, a_ref, al_ref, w3_ref,
                  o_ref, o2_ref):
    dinv = lax.rsqrt(_deg_of(degp_ref[...]))
    gs = []
    for p in (0, 1):
        g = dinv[:, None] * (acc_ref[p] + xs_ref[p]) + b_ref[...][p]
        g = jnp.where(g >= 0, g, a_ref[...][p] * g)
        nrm = jnp.sqrt(jnp.sum(g * g, -1, keepdims=True))
        gs.append(g / jnp.maximum(nrm, 1e-12))
    a0 = al_ref[0, 0]
    a1 = al_ref[0, 1]
    m = jnp.maximum(a0, a1)
    e0 = jnp.exp(a0 - m)
    e1 = jnp.exp(a1 - m)
    z = (e0 / (e0 + e1)) * gs[0] + (e1 / (e0 + e1)) * gs[1]
    o_ref[...] = z
    for t in range(3):
        zdt = jnp.dot(z, w3_ref[t], preferred_element_type=jnp.float32)
        for c in range(4):
            o2_ref[t * 4 + c] = zdt[:, c * 32:(c + 1) * 32]


def _combine_dec(acc2, xs2, degp, b2, a2, alpha, w3):
    full2 = pl.BlockSpec((2, _BLK, _HID), lambda i: (0, i, 0))
    return pl.pallas_call(
        _combine_body,
        grid=(_NB,),
        in_specs=[full2, full2, _DEGSPEC1,
                  pl.BlockSpec((2, _HID), lambda i: (0, 0)),
                  pl.BlockSpec((2, _HID), lambda i: (0, 0)),
                  pl.BlockSpec((1, 2), lambda i: (0, 0)),
                  pl.BlockSpec((3, _HID, _HID), lambda i: (0, 0, 0))],
        out_specs=[pl.BlockSpec((_BLK, _HID), lambda i: (i, 0)),
                   pl.BlockSpec((12, _BLK, 32), lambda i: (0, i, 0))],
        out_shape=[jax.ShapeDtypeStruct((_N, _HID), jnp.float32),
                   jax.ShapeDtypeStruct((12, _N, 32), jnp.float32)],
    )(acc2, xs2, degp, b2, a2, alpha, w3)


def _addp_body(p_ref, o_ref):
    o_ref[...] = p_ref[0] + p_ref[1]


def _add_partials(parts):
    return pl.pallas_call(
        _addp_body,
        grid=(_NB,),
        in_specs=[pl.BlockSpec((2, _BLK, _HID), lambda i: (0, i, 0))],
        out_specs=pl.BlockSpec((_BLK, _HID), lambda i: (i, 0)),
        out_shape=jax.ShapeDtypeStruct((_N, _HID), jnp.float32),
    )(parts)


_CH = 80
_NROW_T = 632
_NPAD = 16 * _NROW_T
_ZR = _NROW_T


def _zero_rows(zb, d):
    zeros16 = jnp.zeros((16,), jnp.float32)

    def zrow(r, c):
        for cc in range(d // 16):
            zb[r, pl.ds(cc * 16, 16)] = zeros16
        return c

    lax.fori_loop(0, _ZR, zrow, 0)


def _zero_acc_and_barrier(zb, acc, row0):
    pltpu.sync_copy(zb, acc.at[pl.ds(row0, _ZR)])
    plsc.subcore_barrier()


def _make_scatter(d, tpc, scaled=True):
    pt = _E // 16
    nch = pt // _CH

    @functools.partial(
        pl.kernel,
        out_type=jax.ShapeDtypeStruct((2 * tpc, _NPAD, d), jnp.float32),
        mesh=plsc.VectorSubcoreMesh(core_axis_name="c", subcore_axis_name="s"),
        compiler_params=pltpu.CompilerParams(use_tc_tiling_on_sc=False),
        scratch_types=[
            pltpu.VMEM((nch, _CH), jnp.int32),
            pltpu.VMEM((nch, _CH), jnp.int32),
            pltpu.VMEM((nch, _CH), jnp.float32),
            pltpu.VMEM((2, _CH, d), jnp.float32),
            pltpu.VMEM((_ZR, d), jnp.float32),
            pltpu.VMEM_SHARED((_NPAD, d), jnp.float32),
            pltpu.SemaphoreType.DMA,
            pltpu.SemaphoreType.DMA,
        ],
    )
    def k(*args):
        tbls = args[:2 * tpc]
        (gi_h, si_h, val_h, out_h,
         gi_v, si_v, val_v, rows_v, zb, acc, sem0, sem1) = args[2 * tpc:]
        sems = (sem0, sem1)
        cid = lax.axis_index("c")
        sid = lax.axis_index("s")
        row0 = sid * _NROW_T
        _zero_rows(zb, d)
        pltpu.sync_copy(gi_h.at[pl.ds(sid * nch, nch)], gi_v)
        pltpu.sync_copy(si_h.at[pl.ds(sid * nch, nch)], si_v)
        pltpu.sync_copy(val_h.at[pl.ds(sid * nch, nch)], val_v)

        def run(tbl, tglob):
            pltpu.sync_copy(zb, acc.at[pl.ds(row0, _ZR)])
            plsc.subcore_barrier()

            pltpu.async_copy(tbl.at[gi_v.at[0]], rows_v.at[0], sems[0])

            def step(jj, b, issue_next):
                if issue_next:
                    pltpu.async_copy(tbl.at[gi_v.at[jj + 1]],
                                     rows_v.at[1 - b], sems[1 - b])
                pltpu.make_async_copy(tbl.at[gi_v.at[jj]],
                                      rows_v.at[b], sems[b]).wait()
                if scaled:
                    def sgrp(g, c2, b=b, jj=jj):
                        v16 = val_v[jj, pl.ds(g * 16, 16)]
                        for r in range(16):
                            s = v16[r]
                            for cc in range(d // 16):
                                sl = pl.ds(cc * 16, 16)
                                rows_v[b, g * 16 + r, sl] = \
                                    rows_v[b, g * 16 + r, sl] * s
                        return c2

                    lax.fori_loop(0, _CH // 16, sgrp, 0)
                pltpu.sync_copy(rows_v.at[b], acc.at[si_v.at[jj]], add=True)

            def body(jh, c):
                for b in range(2):
                    step(jh * 2 + b, b, True)
                return c

            lax.fori_loop(0, nch // 2 - 1, body, 0)
            step(nch - 2, 0, True)
            step(nch - 1, 1, False)
            plsc.subcore_barrier()
            pltpu.sync_copy(acc.at[pl.ds(row0, _ZR)],
                            out_h.at[tglob, pl.ds(row0, _ZR)])

        for tloc in range(tpc):
            @pl.when(cid == 0)
            def _(tloc=tloc):
                run(tbls[tloc], tloc)

            @pl.when(cid == 1)
            def _(tloc=tloc):
                run(tbls[tpc + tloc], tpc + tloc)

    return k


def _make_deg_kernel():
    @functools.partial(
        pl.kernel,
        out_type=jax.ShapeDtypeStruct((2, _NPAD, 16), jnp.float32),
        mesh=plsc.VectorSubcoreMesh(core_axis_name="c", subcore_axis_name="s"),
        compiler_params=pltpu.CompilerParams(use_tc_tiling_on_sc=False),
        scratch_types=[
            pltpu.VMEM((_CH,), jnp.int32),
            pltpu.VMEM((_CH, 16), jnp.float32),
            pltpu.VMEM((_ZR, 16), jnp.float32),
            pltpu.VMEM_SHARED((_NPAD, 16), jnp.float32),
        ],
    )
    def _deg_kernel(dst_h, out_h, dst_v, ones_v, zb, acc):
        cid = lax.axis_index("c")
        sid = lax.axis_index("s")
        row0 = sid * _NROW_T
        ones16 = jnp.ones((16,), jnp.float32)

        def orow(r, c):
            ones_v[r, pl.ds(0, 16)] = ones16
            return c

        lax.fori_loop(0, _CH, orow, 0)
        _zero_rows(zb, 16)
        _zero_acc_and_barrier(zb, acc, row0)

        pt = _E // 32
        nch = pt // _CH

        def body(j, c):
            base = (cid * 16 + sid) * pt + j * _CH
            pltpu.sync_copy(dst_h.at[pl.ds(base, _CH)], dst_v)
            pltpu.sync_copy(ones_v, acc.at[dst_v], add=True)
            return c

        lax.fori_loop(0, nch, body, 0)
        plsc.subcore_barrier()
        pltpu.sync_copy(acc.at[pl.ds(row0, _ZR)],
                        out_h.at[cid, pl.ds(row0, _ZR)])

    return _deg_kernel


_SC_CACHE = {}


def _get_sc(name):
    if name not in _SC_CACHE:
        _SC_CACHE['deg'] = _make_deg_kernel()
        _SC_CACHE['conv'] = _make_scatter(32, tpc=4, scaled=False)
        _SC_CACHE['dec'] = _make_scatter(32, tpc=6)
    return _SC_CACHE[name]


def _stk1(arrs):
    return jnp.stack(arrs)[:, None, :]

def _encode(x, p, projs, egs):
    w0 = jnp.stack([jnp.pad(pe['fc0_w'], ((0, _IN1 - pe['fc0_w'].shape[0]),
                                          (0, 0))) for pe in p])
    b0 = _stk1([pe['fc0_b'] for pe in p])
    g0 = _stk1([pe['ln0_g'] for pe in p])
    bb0 = _stk1([pe['ln0_b'] for pe in p])
    z = _prologue(x, w0, b0, g0, bb0)
    layers = [z]
    for i in range(_NLAYERS):
        cs = [pe['conv%d' % i] for pe in p]
        pj = jnp.stack([projs[0][i], projs[1][i]])
        eg = jnp.concatenate([egs[0][i], egs[1][i]], axis=0)
        wk = jnp.stack([c['Wk_w'] for c in cs])
        bk = _stk1([c['Wk_b'] for c in cs])
        wv = jnp.stack([c['Wv_w'] for c in cs])
        bv = _stk1([c['Wv_b'] for c in cs])
        wq = jnp.stack([c['Wq_w'] for c in cs])
        bq = _stk1([c['Wq_b'] for c in cs])
        wo = jnp.stack([c['Wo_w'] for c in cs])
        bo = _stk1([c['Wo_b'] for c in cs])
        lg = _stk1([pe['ln%d_g' % (i + 1)] for pe in p])
        lb = _stk1([pe['ln%d_b' % (i + 1)] for pe in p])
        z = _layer(z, eg, wk, bk, wv, bv, pj, wq, bq, wo, bo, lg, lb)
        layers.append(z)
    return layers


def kernel(x1, x2, edge_index, adj_values, params):
    projs_all, egs_all = _rand_consts()
    projs = (projs_all[:_NLAYERS], projs_all[_NLAYERS:])
    egs = (egs_all[:_NLAYERS], egs_all[_NLAYERS:])

    src = edge_index[0]
    dst = edge_index[1]
    src2 = src.reshape(_E // _CH, _CH)
    dst2 = dst.reshape(_E // _CH, _CH)
    adj2 = adj_values.reshape(_E // _CH, _CH)

    degp = _get_sc('deg')(dst)

    x = jnp.concatenate(
        [x1, jnp.pad(x2, ((0, 0), (0, _IN1 - _IN2)))], axis=0)
    encs = (params['enc1'], params['enc2'])
    layers = _encode(x, encs, projs, egs)

    f = params['fus']
    sc_conv = _get_sc('conv')
    sc_dec = _get_sc('dec')
    ones2 = jnp.ones((_E // _CH, _CH), jnp.float32)

    def split32(a):
        return a.reshape(2, _N, 4, 32).transpose(0, 2, 1, 3).reshape(8, _N, 32)

    def join32(a8):
        return a8.reshape(2, 4, _NPAD, 32).transpose(0, 2, 1, 3).reshape(
            2, _NPAD, 128)

    w1 = jnp.stack([pe['fc1_w'] for pe in encs])
    b1 = _stk1([pe['fc1_b'] for pe in encs])
    w12 = jnp.stack([f['c1_w'], f['c2_w']])
    xs = _epi_pre(layers[0], layers[1], layers[2], layers[3], w1, b1,
                  degp, w12)
    xs8 = split32(xs)
    acc1 = join32(sc_conv(*(xs8[i] for i in range(8)), src2, dst2, ones2))

    b12 = _stk1([f['c1_b'], f['c2_b']])
    a13 = _stk1([f['prelu1'], f['prelu3']])
    w34 = jnp.stack([f['c3_w'], f['c4_w']])
    xs2 = _gcn_mid(acc1, xs, degp, b12, a13, w34)
    xs28 = split32(xs2)
    acc2 = join32(sc_conv(*(xs28[i] for i in range(8)), src2, dst2, ones2))

    b34 = jnp.stack([f['c3_b'], f['c4_b']])
    a24 = jnp.stack([f['prelu2'], f['prelu4']])
    w3 = jnp.stack([params['dec1_w'][:, :128], params['dec1_w'][:, 128:],
                    params['dec2_w']])
    z, zd12 = _combine_dec(acc2, xs2, degp, b34, a24,
                           f['alpha'].reshape(1, 2), w3)

    accd = sc_dec(*(zd12[i] for i in range(12)), dst2, src2, adj2)
    r1 = jnp.concatenate([accd[i, :_N] for i in range(8)], axis=1)
    r2 = jnp.concatenate([accd[i, :_N] for i in range(8, 12)], axis=1)

    return (z, r1, r2)


# speedup vs baseline: 1.9607x; 1.3078x over previous
import functools
import math

import jax
import jax.numpy as jnp
import numpy as np
from jax import lax
from jax.experimental import pallas as pl
from jax.experimental.pallas import tpu as pltpu
from jax.experimental.pallas import tpu_sc as plsc

_N = 10000
_E = 320000
_IN1, _OUT1 = 256, 128
_IN2, _OUT2 = 128, 128
_HID = 128
_NLAYERS = 3
_MFEAT = 30
_KG = 10
_DN = 1.0 / math.sqrt(math.sqrt(128.0))

_BLK = 2000
_NB = _N // _BLK


def _rand_consts_raw():
    rk = jax.random.key(42)
    projs, egs = [], []
    for enc in (1, 2):
        key = jax.random.fold_in(rk, enc)
        for i in range(_NLAYERS):
            lk = jax.random.fold_in(key, i)
            kp, kg = jax.random.split(lk)
            k1, k2 = jax.random.split(kp)
            q, _ = jnp.linalg.qr(jax.random.normal(k1, (_HID, _HID)))
            mult = jnp.linalg.norm(jax.random.normal(k2, (_MFEAT, _HID)), axis=1)
            proj = mult[:, None] * q[:_MFEAT]
            g = -jnp.log(jax.random.exponential(kg, (_N, 1, _KG)))
            projs.append(proj.astype(jnp.float32))
            egs.append(jnp.exp(g).reshape(_N, _KG).astype(jnp.float32))
    return projs, egs


@functools.lru_cache(maxsize=None)
def _rand_consts_host():
    with jax.ensure_compile_time_eval(), \
         jax.default_device(jax.local_devices(backend="cpu")[0]):
        projs, egs = _rand_consts_raw()
        return ([np.asarray(p) for p in projs], [np.asarray(e) for e in egs])


def _rand_consts():
    try:
        return _rand_consts_host()
    except Exception:
        return _rand_consts_raw()


def _ln_elu(z, g, b):
    mu = jnp.mean(z, -1, keepdims=True)
    zc = z - mu
    var = jnp.mean(zc * zc, -1, keepdims=True)
    z = zc * lax.rsqrt(var + 1e-5) * g + b
    return jnp.where(z > 0, z, jnp.exp(jnp.minimum(z, 0.0)) - 1.0)


def _row_spec(width):
    return pl.BlockSpec((_BLK, width), lambda b, i: (b * _NB + i, 0))


def _wspec(shape):
    nz = len(shape)
    return pl.BlockSpec((1,) + shape, lambda b, i, nz=nz: (b,) + (0,) * nz)


def _prologue_body(x_ref, w_ref, b_ref, g_ref, bb_ref, o_ref):
    z = jnp.dot(x_ref[...], w_ref[0], preferred_element_type=jnp.float32)
    z = z + b_ref[0, 0]
    o_ref[...] = _ln_elu(z, g_ref[0, 0], bb_ref[0, 0])


def _prologue(x, w, b, g, bb):
    return pl.pallas_call(
        _prologue_body,
        grid=(2, _NB),
        in_specs=[
            _row_spec(x.shape[1]),
            _wspec(w.shape[1:]),
            _wspec((1, _HID)),
            _wspec((1, _HID)),
            _wspec((1, _HID)),
        ],
        out_specs=_row_spec(_HID),
        out_shape=jax.ShapeDtypeStruct((2 * _N, _HID), jnp.float32),
    )(x, w, b, g, bb)


def _layer_body(z_ref, eg_ref, wk_ref, bk_ref, wv_ref, bv_ref, pj_ref,
                wq_ref, bq_ref, wo_ref, bo_ref, lg_ref, lb_ref, o_ref,
                s1_ref, t_ref, s1s_ref, ts_ref, gm_ref):
    ph = pl.program_id(1)
    i = pl.program_id(2)

    @pl.when((ph == 0) & (i == 0))
    def _():
        s1_ref[...] = jnp.zeros_like(s1_ref)
        t_ref[...] = jnp.zeros_like(t_ref)
        s1s_ref[...] = jnp.zeros_like(s1s_ref)
        ts_ref[...] = jnp.zeros_like(ts_ref)
        gm_ref[...] = jnp.full(gm_ref.shape, -jnp.inf, jnp.float32)

    z = z_ref[...]

    @pl.when(ph == 0)
    def _():
        zk = (jnp.dot(z, wk_ref[0], preferred_element_type=jnp.float32)
              + bk_ref[0, 0]) * _DN
        dd = lax.dot_general(zk, pj_ref[0], (((1,), (1,)), ((), ())),
                             preferred_element_type=jnp.float32)
        diag = 0.5 * jnp.sum(zk * zk, -1, keepdims=True)
        p = jnp.exp(dd - diag)
        v = (jnp.dot(z, wv_ref[0], preferred_element_type=jnp.float32)
             + bv_ref[0, 0])
        eg = eg_ref[...]
        for k in range(_KG):
            a = p * eg[:, k:k + 1]
            s1k = lax.dot_general(a, v, (((0,), (0,)), ((), ())),
                                  preferred_element_type=jnp.float32)
            s1_ref[k] += s1k
        t_ref[...] += lax.dot_general(eg, v, (((0,), (0,)), ((), ())),
                                      preferred_element_type=jnp.float32)
        s1s_ref[...] += lax.dot_general(eg, p, (((0,), (0,)), ((), ())),
                                        preferred_element_type=jnp.float32)
        ts_ref[...] += jnp.sum(eg, axis=0).reshape(1, _KG)
        gm_ref[...] = jnp.maximum(gm_ref[...],
                                  jnp.reshape(jnp.max(dd), (1, 1)))

    @pl.when(ph == 1)
    def _():
        zq = (jnp.dot(z, wq_ref[0], preferred_element_type=jnp.float32)
              + bq_ref[0, 0]) * _DN
        dd = lax.dot_general(zq, pj_ref[0], (((1,), (1,)), ((), ())),
                             preferred_element_type=jnp.float32)
        diag = 0.5 * jnp.sum(zq * zq, -1, keepdims=True)
        rmax = jnp.max(dd, -1, keepdims=True)
        qp = jnp.exp(dd - diag - rmax) + 1e-6
        em = jnp.exp(-gm_ref[0, 0])
        acc = jnp.zeros((z.shape[0], _HID), jnp.float32)
        for k in range(_KG):
            kvsk = em * s1_ref[k] + 1e-6 * t_ref[k][None, :]
            kssk = em * s1s_ref[k] + 1e-6 * ts_ref[0, k]
            num = jnp.dot(qp, kvsk, preferred_element_type=jnp.float32)
            den = jnp.sum(qp * kssk[None, :], -1, keepdims=True)
            acc = acc + num / den
        z_att = acc * (1.0 / _KG)
        out = jnp.dot(z_att, wo_ref[0], preferred_element_type=jnp.float32)
        out = out + bo_ref[0, 0] + z
        o_ref[...] = _ln_elu(out, lg_ref[0, 0], lb_ref[0, 0])


def _layer(z, eg, wk, bk, wv, bv, pj, wq, bq, wo, bo, lg, lb):
    rspec = lambda w: pl.BlockSpec((_BLK, w),
                                   lambda b, ph, i: (b * _NB + i, 0))
    wspec = lambda shape: pl.BlockSpec(
        (1,) + shape, lambda b, ph, i, nz=len(shape): (b,) + (0,) * nz)
    return pl.pallas_call(
        _layer_body,
        grid=(2, 2, _NB),
        in_specs=[
            rspec(_HID), rspec(_KG),
            wspec((_HID, _HID)), wspec((1, _HID)),
            wspec((_HID, _HID)), wspec((1, _HID)),
            wspec((_MFEAT, _HID)),
            wspec((_HID, _HID)), wspec((1, _HID)),
            wspec((_HID, _HID)), wspec((1, _HID)),
            wspec((1, _HID)), wspec((1, _HID)),
        ],
        out_specs=pl.BlockSpec((_BLK, _HID),
                               lambda b, ph, i: (b * _NB + i * ph, 0)),
        out_shape=jax.ShapeDtypeStruct((2 * _N, _HID), jnp.float32),
        scratch_shapes=[
            pltpu.VMEM((_KG, _MFEAT, _HID), jnp.float32),
            pltpu.VMEM((_KG, _HID), jnp.float32),
            pltpu.VMEM((_KG, _MFEAT), jnp.float32),
            pltpu.VMEM((1, _KG), jnp.float32),
            pltpu.VMEM((1, 1), jnp.float32),
        ],
    )(z, eg, wk, bk, wv, bv, pj, wq, bq, wo, bo, lg, lb)


def _deg_of(degp_blk):
    return degp_blk[0, :, 0] + degp_blk[1, :, 0] + 1.0


_PSPEC = pl.BlockSpec((1, _BLK, _HID), lambda p, i: (p, i, 0))
_DEGSPEC2 = pl.BlockSpec((2, _BLK, 16), lambda p, i: (0, i, 0))
_DEGSPEC1 = pl.BlockSpec((2, _BLK, 16), lambda i: (0, i, 0))


def _epi_pre_body(z0_ref, z1_ref, z2_ref, z3_ref, w_ref, b_ref, degp_ref,
                  wc_ref, o_ref):
    w = w_ref[0]
    out = jnp.dot(z0_ref[...], w[0:128], preferred_element_type=jnp.float32)
    out += jnp.dot(z1_ref[...], w[128:256], preferred_element_type=jnp.float32)
    out += jnp.dot(z2_ref[...], w[256:384], preferred_element_type=jnp.float32)
    out += jnp.dot(z3_ref[...], w[384:512], preferred_element_type=jnp.float32)
    e = out + b_ref[0, 0]
    dinv = lax.rsqrt(_deg_of(degp_ref[...]))
    o_ref[0] = dinv[:, None] * jnp.dot(e, wc_ref[0],
                                       preferred_element_type=jnp.float32)


def _epi_pre(z0, z1, z2, z3, w, b, degp, wc):
    return pl.pallas_call(
        _epi_pre_body,
        grid=(2, _NB),
        in_specs=[_row_spec(_HID)] * 4 + [_wspec((4 * _HID, _HID)),
                                          _wspec((1, _HID)), _DEGSPEC2,
                                          _wspec((_HID, _HID))],
        out_specs=_PSPEC,
        out_shape=jax.ShapeDtypeStruct((2, _N, _HID), jnp.float32),
    )(z0, z1, z2, z3, w, b, degp, wc)


def _gcn_mid_body(acc_ref, xs_ref, degp_ref, b_ref, a_ref, w2_ref, o_ref):
    dinv = lax.rsqrt(_deg_of(degp_ref[...]))
    h = dinv[:, None] * (acc_ref[0] + xs_ref[0]) + b_ref[0, 0]
    h = jnp.where(h >= 0, h, a_ref[0, 0] * h)
    o_ref[0] = dinv[:, None] * jnp.dot(h, w2_ref[0],
                                       preferred_element_type=jnp.float32)


def _gcn_mid(acc, xs, degp, b, a, w2):
    return pl.pallas_call(
        _gcn_mid_body,
        grid=(2, _NB),
        in_specs=[_PSPEC, _PSPEC, _DEGSPEC2, _wspec((1, _HID)), _wspec((1, _HID)),
                  _wspec((_HID, _HID))],
        out_specs=_PSPEC,
        out_shape=jax.ShapeDtypeStruct((2, _N, _HID), jnp.float32),
    )(acc, xs, degp, b, a, w2)


def _combine_body(acc_ref, xs_ref, degp_ref, b_ref, a_ref, al_ref,
                  o_ref, o2_ref):
    dinv = lax.rsqrt(_deg_of(degp_ref[...]))
    gs = []
    for p in (0, 1):
        g = dinv[:, None] * (acc_ref[p] + xs_ref[p]) + b_ref[...][p]
        g = jnp.where(g >= 0, g, a_ref[...][p] * g)
        nrm = jnp.sqrt(jnp.sum(g * g, -1, keepdims=True))
        gs.append(g / jnp.maximum(nrm, 1e-12))
    a0 = al_ref[0, 0]
    a1 = al_ref[0, 1]
    m = jnp.maximum(a0, a1)
    e0 = jnp.exp(a0 - m)
    e1 = jnp.exp(a1 - m)
    z = (e0 / (e0 + e1)) * gs[0] + (e1 / (e0 + e1)) * gs[1]
    o_ref[...] = z
    for c in range(4):
        o2_ref[c] = z[:, c * 32:(c + 1) * 32]


def _combine_dec(acc2, xs2, degp, b2, a2, alpha):
    full2 = pl.BlockSpec((2, _BLK, _HID), lambda i: (0, i, 0))
    return pl.pallas_call(
        _combine_body,
        grid=(_NB,),
        in_specs=[full2, full2, _DEGSPEC1,
                  pl.BlockSpec((2, _HID), lambda i: (0, 0)),
                  pl.BlockSpec((2, _HID), lambda i: (0, 0)),
                  pl.BlockSpec((1, 2), lambda i: (0, 0))],
        out_specs=[pl.BlockSpec((_BLK, _HID), lambda i: (i, 0)),
                   pl.BlockSpec((4, _BLK, 32), lambda i: (0, i, 0))],
        out_shape=[jax.ShapeDtypeStruct((_N, _HID), jnp.float32),
                   jax.ShapeDtypeStruct((4, _N, 32), jnp.float32)],
    )(acc2, xs2, degp, b2, a2, alpha)


def _dec_mm_body(z_ref, w1_ref, w2_ref, o1_ref, o2_ref):
    o1_ref[...] = jnp.dot(z_ref[...], w1_ref[...],
                          preferred_element_type=jnp.float32)
    o2_ref[...] = jnp.dot(z_ref[...], w2_ref[...],
                          preferred_element_type=jnp.float32)


def _dec_mm(zagg, w1, w2):
    return pl.pallas_call(
        _dec_mm_body,
        grid=(_NB,),
        in_specs=[pl.BlockSpec((_BLK, _HID), lambda i: (i, 0)),
                  pl.BlockSpec((_HID, 256), lambda i: (0, 0)),
                  pl.BlockSpec((_HID, _HID), lambda i: (0, 0))],
        out_specs=[pl.BlockSpec((_BLK, 256), lambda i: (i, 0)),
                   pl.BlockSpec((_BLK, _HID), lambda i: (i, 0))],
        out_shape=[jax.ShapeDtypeStruct((_N, 256), jnp.float32),
                   jax.ShapeDtypeStruct((_N, _HID), jnp.float32)],
    )(zagg, w1, w2)


def _addp_body(p_ref, o_ref):
    o_ref[...] = p_ref[0] + p_ref[1]


def _add_partials(parts):
    return pl.pallas_call(
        _addp_body,
        grid=(_NB,),
        in_specs=[pl.BlockSpec((2, _BLK, _HID), lambda i: (0, i, 0))],
        out_specs=pl.BlockSpec((_BLK, _HID), lambda i: (i, 0)),
        out_shape=jax.ShapeDtypeStruct((_N, _HID), jnp.float32),
    )(parts)


_CH = 80
_NROW_T = 632
_NPAD = 16 * _NROW_T
_ZR = _NROW_T


def _zero_rows(zb, d):
    zeros16 = jnp.zeros((16,), jnp.float32)

    def zrow(r, c):
        for cc in range(d // 16):
            zb[r, pl.ds(cc * 16, 16)] = zeros16
        return c

    lax.fori_loop(0, _ZR, zrow, 0)


def _zero_acc_and_barrier(zb, acc, row0):
    pltpu.sync_copy(zb, acc.at[pl.ds(row0, _ZR)])
    plsc.subcore_barrier()


def _make_scatter(d, tpc, scaled=True):
    pt = _E // 16
    nch = pt // _CH

    @functools.partial(
        pl.kernel,
        out_type=jax.ShapeDtypeStruct((2 * tpc, _NPAD, d), jnp.float32),
        mesh=plsc.VectorSubcoreMesh(core_axis_name="c", subcore_axis_name="s"),
        compiler_params=pltpu.CompilerParams(use_tc_tiling_on_sc=False),
        scratch_types=[
            pltpu.VMEM((nch, _CH), jnp.int32),
            pltpu.VMEM((nch, _CH), jnp.int32),
            pltpu.VMEM((nch, _CH), jnp.float32),
            pltpu.VMEM((2, _CH, d), jnp.float32),
            pltpu.VMEM((_ZR, d), jnp.float32),
            pltpu.VMEM_SHARED((_NPAD, d), jnp.float32),
            pltpu.SemaphoreType.DMA,
            pltpu.SemaphoreType.DMA,
        ],
    )
    def k(*args):
        tbls = args[:2 * tpc]
        (gi_h, si_h, val_h, out_h,
         gi_v, si_v, val_v, rows_v, zb, acc, sem0, sem1) = args[2 * tpc:]
        sems = (sem0, sem1)
        cid = lax.axis_index("c")
        sid = lax.axis_index("s")
        row0 = sid * _NROW_T
        _zero_rows(zb, d)
        pltpu.sync_copy(gi_h.at[pl.ds(sid * nch, nch)], gi_v)
        pltpu.sync_copy(si_h.at[pl.ds(sid * nch, nch)], si_v)
        pltpu.sync_copy(val_h.at[pl.ds(sid * nch, nch)], val_v)

        def run(tbl, tglob):
            pltpu.sync_copy(zb, acc.at[pl.ds(row0, _ZR)])
            plsc.subcore_barrier()

            pltpu.async_copy(tbl.at[gi_v.at[0]], rows_v.at[0], sems[0])

            def step(jj, b, issue_next):
                if issue_next:
                    pltpu.async_copy(tbl.at[gi_v.at[jj + 1]],
                                     rows_v.at[1 - b], sems[1 - b])
                pltpu.make_async_copy(tbl.at[gi_v.at[jj]],
                                      rows_v.at[b], sems[b]).wait()
                if scaled:
                    def sgrp(g, c2, b=b, jj=jj):
                        v16 = val_v[jj, pl.ds(g * 16, 16)]
                        for r in range(16):
                            s = v16[r]
                            for cc in range(d // 16):
                                sl = pl.ds(cc * 16, 16)
                                rows_v[b, g * 16 + r, sl] = \
                                    rows_v[b, g * 16 + r, sl] * s
                        return c2

                    lax.fori_loop(0, _CH // 16, sgrp, 0)
                pltpu.sync_copy(rows_v.at[b], acc.at[si_v.at[jj]], add=True)

            def body(jh, c):
                for b in range(2):
                    step(jh * 2 + b, b, True)
                return c

            lax.fori_loop(0, nch // 2 - 1, body, 0)
            step(nch - 2, 0, True)
            step(nch - 1, 1, False)
            plsc.subcore_barrier()
            pltpu.sync_copy(acc.at[pl.ds(row0, _ZR)],
                            out_h.at[tglob, pl.ds(row0, _ZR)])

        for tloc in range(tpc):
            @pl.when(cid == 0)
            def _(tloc=tloc):
                run(tbls[tloc], tloc)

            @pl.when(cid == 1)
            def _(tloc=tloc):
                run(tbls[tpc + tloc], tpc + tloc)

    return k


def _make_deg_kernel():
    @functools.partial(
        pl.kernel,
        out_type=jax.ShapeDtypeStruct((2, _NPAD, 16), jnp.float32),
        mesh=plsc.VectorSubcoreMesh(core_axis_name="c", subcore_axis_name="s"),
        compiler_params=pltpu.CompilerParams(use_tc_tiling_on_sc=False),
        scratch_types=[
            pltpu.VMEM((_CH,), jnp.int32),
            pltpu.VMEM((_CH, 16), jnp.float32),
            pltpu.VMEM((_ZR, 16), jnp.float32),
            pltpu.VMEM_SHARED((_NPAD, 16), jnp.float32),
        ],
    )
    def _deg_kernel(dst_h, out_h, dst_v, ones_v, zb, acc):
        cid = lax.axis_index("c")
        sid = lax.axis_index("s")
        row0 = sid * _NROW_T
        ones16 = jnp.ones((16,), jnp.float32)

        def orow(r, c):
            ones_v[r, pl.ds(0, 16)] = ones16
            return c

        lax.fori_loop(0, _CH, orow, 0)
        _zero_rows(zb, 16)
        _zero_acc_and_barrier(zb, acc, row0)

        pt = _E // 32
        nch = pt // _CH

        def body(j, c):
            base = (cid * 16 + sid) * pt + j * _CH
            pltpu.sync_copy(dst_h.at[pl.ds(base, _CH)], dst_v)
            pltpu.sync_copy(ones_v, acc.at[dst_v], add=True)
            return c

        lax.fori_loop(0, nch, body, 0)
        plsc.subcore_barrier()
        pltpu.sync_copy(acc.at[pl.ds(row0, _ZR)],
                        out_h.at[cid, pl.ds(row0, _ZR)])

    return _deg_kernel


_SC_CACHE = {}


def _get_sc(name):
    if name not in _SC_CACHE:
        _SC_CACHE['deg'] = _make_deg_kernel()
        _SC_CACHE['conv'] = _make_scatter(32, tpc=4, scaled=False)
        _SC_CACHE['dec'] = _make_scatter(32, tpc=2)
    return _SC_CACHE[name]


def _stk1(arrs):
    return jnp.stack(arrs)[:, None, :]

def _encode(x, p, projs, egs):
    w0 = jnp.stack([jnp.pad(pe['fc0_w'], ((0, _IN1 - pe['fc0_w'].shape[0]),
                                          (0, 0))) for pe in p])
    b0 = _stk1([pe['fc0_b'] for pe in p])
    g0 = _stk1([pe['ln0_g'] for pe in p])
    bb0 = _stk1([pe['ln0_b'] for pe in p])
    z = _prologue(x, w0, b0, g0, bb0)
    layers = [z]
    for i in range(_NLAYERS):
        cs = [pe['conv%d' % i] for pe in p]
        pj = jnp.stack([projs[0][i], projs[1][i]])
        eg = jnp.concatenate([egs[0][i], egs[1][i]], axis=0)
        wk = jnp.stack([c['Wk_w'] for c in cs])
        bk = _stk1([c['Wk_b'] for c in cs])
        wv = jnp.stack([c['Wv_w'] for c in cs])
        bv = _stk1([c['Wv_b'] for c in cs])
        wq = jnp.stack([c['Wq_w'] for c in cs])
        bq = _stk1([c['Wq_b'] for c in cs])
        wo = jnp.stack([c['Wo_w'] for c in cs])
        bo = _stk1([c['Wo_b'] for c in cs])
        lg = _stk1([pe['ln%d_g' % (i + 1)] for pe in p])
        lb = _stk1([pe['ln%d_b' % (i + 1)] for pe in p])
        z = _layer(z, eg, wk, bk, wv, bv, pj, wq, bq, wo, bo, lg, lb)
        layers.append(z)
    return layers


def kernel(x1, x2, edge_index, adj_values, params):
    projs_all, egs_all = _rand_consts()
    projs = (projs_all[:_NLAYERS], projs_all[_NLAYERS:])
    egs = (egs_all[:_NLAYERS], egs_all[_NLAYERS:])

    src = edge_index[0]
    dst = edge_index[1]
    src2 = src.reshape(_E // _CH, _CH)
    dst2 = dst.reshape(_E // _CH, _CH)
    adj2 = adj_values.reshape(_E // _CH, _CH)

    degp = _get_sc('deg')(dst)

    x = jnp.concatenate(
        [x1, jnp.pad(x2, ((0, 0), (0, _IN1 - _IN2)))], axis=0)
    encs = (params['enc1'], params['enc2'])
    layers = _encode(x, encs, projs, egs)

    f = params['fus']
    sc_conv = _get_sc('conv')
    sc_dec = _get_sc('dec')
    ones2 = jnp.ones((_E // _CH, _CH), jnp.float32)

    def split32(a):
        return a.reshape(2, _N, 4, 32).transpose(0, 2, 1, 3).reshape(8, _N, 32)

    def join32(a8):
        return a8.reshape(2, 4, _NPAD, 32).transpose(0, 2, 1, 3).reshape(
            2, _NPAD, 128)

    w1 = jnp.stack([pe['fc1_w'] for pe in encs])
    b1 = _stk1([pe['fc1_b'] for pe in encs])
    w12 = jnp.stack([f['c1_w'], f['c2_w']])
    xs = _epi_pre(layers[0], layers[1], layers[2], layers[3], w1, b1,
                  degp, w12)
    xs8 = split32(xs)
    acc1 = join32(sc_conv(*(xs8[i] for i in range(8)), src2, dst2, ones2))

    b12 = _stk1([f['c1_b'], f['c2_b']])
    a13 = _stk1([f['prelu1'], f['prelu3']])
    w34 = jnp.stack([f['c3_w'], f['c4_w']])
    xs2 = _gcn_mid(acc1, xs, degp, b12, a13, w34)
    xs28 = split32(xs2)
    acc2 = join32(sc_conv(*(xs28[i] for i in range(8)), src2, dst2, ones2))

    b34 = jnp.stack([f['c3_b'], f['c4_b']])
    a24 = jnp.stack([f['prelu2'], f['prelu4']])
    z, z4 = _combine_dec(acc2, xs2, degp, b34, a24, f['alpha'].reshape(1, 2))

    accd = sc_dec(*(z4[i] for i in range(4)), dst2, src2, adj2)
    zagg = jnp.concatenate([accd[i, :_N] for i in range(4)], axis=1)
    r1, r2 = _dec_mm(zagg, params['dec1_w'], params['dec2_w'])

    return (z, r1, r2)
